# Initial kernel scaffold; baseline (speedup 1.0000x reference)
import jax
import jax.numpy as jnp
from jax.experimental import pallas as pl


def kernel(x, e, pe, edge_index, W_pe, b_pe, W1e, b1e, W2e, b2e, A, B, C, U, V, Wp1, bp1, Wp2, bp2):
    raise NotImplementedError("write your pallas kernel here")


# speedup vs baseline: 1.4281x; 1.4281x over previous
import functools

import jax
import jax.numpy as jnp
from jax import lax
from jax.experimental import pallas as pl
from jax.experimental.pallas import tpu as pltpu
from jax.experimental.pallas import tpu_sc as plsc

F32 = jnp.float32

N = 10000
E = 320000
D = 128
H = 64
NP = 10240
L = 3

NS = 16
NC = 2
CK = 80
EPT = E // NS
EPW = E // (NS * NC)
RPT = NP // NS

_mesh = plsc.VectorSubcoreMesh(core_axis_name="c", subcore_axis_name="s")


def _sigmoid(x):
    return 1.0 / (1.0 + jnp.exp(-x))


def _enc_h_body(pe_ref, wpe_ref, bpe_ref, h_ref):
    h_ref[...] = (
        jnp.dot(pe_ref[...], wpe_ref[...], preferred_element_type=F32)
        + bpe_ref[...]
    )


def _enc_et_body(e_ref, w1_ref, b1_ref, w2_ref, b2_ref, c0_ref, et_ref, etc_ref):
    t = jnp.maximum(
        jnp.dot(e_ref[...], w1_ref[...], preferred_element_type=F32) + b1_ref[...],
        0.0,
    )
    et = jnp.dot(t, w2_ref[...], preferred_element_type=F32) + b2_ref[...]
    et_ref[...] = et
    etc = jnp.dot(et, c0_ref[...], preferred_element_type=F32)
    etc_ref[0] = etc[:, :H]
    etc_ref[1] = etc[:, H:]


def _tables_body(h_ref, etl_ref, a_ref, b_ref, v_ref, u_ref, c_ref,
                 hav_ref, hb2_ref, hu_ref, gl_ref, aggl_ref, etln_ref):
    h = h_ref[...]
    hA = jnp.dot(h, a_ref[...], preferred_element_type=F32)
    hB = jnp.dot(h, b_ref[...], preferred_element_type=F32)
    hV = jnp.dot(h, v_ref[...], preferred_element_type=F32)
    hU = jnp.dot(h, u_ref[...], preferred_element_type=F32)
    eL = hA + hB + jnp.dot(etl_ref[...], c_ref[...], preferred_element_type=F32)
    gL = _sigmoid(eL)
    hav_ref[0] = jnp.concatenate([hA[:, :H], hV[:, :H]], axis=1)
    hav_ref[1] = jnp.concatenate([hA[:, H:], hV[:, H:]], axis=1)
    hb2_ref[0] = hB[:, :H]
    hb2_ref[1] = hB[:, H:]
    hu_ref[...] = hU
    gl_ref[...] = gL
    aggl_ref[...] = gL * hV
    etln_ref[...] = etl_ref[...] + jnp.maximum(eL, 0.0)


def _etc_body(et2_ref, r_ref, c_ref, et2n_ref, etc_ref):
    rfull = jnp.concatenate([r_ref[0], r_ref[1]], axis=1)
    etn = et2_ref[...] + rfull
    et2n_ref[...] = etn
    etc = jnp.dot(etn, c_ref[...], preferred_element_type=F32)
    etc_ref[0] = etc[:, :H]
    etc_ref[1] = etc[:, H:]


def _etp_body(et2_ref, r_ref, w_ref, b_ref, etp_ref):
    etn = et2_ref[...] + jnp.concatenate([r_ref[0], r_ref[1]], axis=1)
    etp_ref[...] = jnp.dot(etn, w_ref[...], preferred_element_type=F32) + b_ref[...]


def _hupd_body(h_ref, hu_ref, aggl_ref, gl_ref, agg_ref, gsum_ref, hn_ref):
    agg = jnp.concatenate([agg_ref[0], agg_ref[1]], axis=1) + aggl_ref[...]
    gs = jnp.concatenate([gsum_ref[0], gsum_ref[1]], axis=1) + gl_ref[...] + 1e-6
    hn_ref[...] = h_ref[...] + jnp.maximum(hu_ref[...] + agg / gs, 0.0)


def _prednode_body(h_ref, wa_ref, wb_ref, hs_ref, hd_ref):
    h = h_ref[...]
    hs_ref[...] = jnp.dot(h, wa_ref[...], preferred_element_type=F32)
    hd_ref[...] = jnp.dot(h, wb_ref[...], preferred_element_type=F32)


def _full(shape):
    return pl.BlockSpec(shape, lambda i: tuple(0 for _ in shape))


def _edge_body(src_ref, dst_ref, hav_ref, hb2_ref, etc_ref, zer_ref,
               r_ref, agg_ref, gsum_ref,
               idx_s, idx_d, idx_g, idx_b, buf_av, buf_b, buf_e, buf_g, buf_m,
               agg_sh, gsum_sh, sem):
    c = lax.axis_index("c")
    s = lax.axis_index("s")
    cn = c * N
    r0 = s * RPT

    pltpu.sync_copy(zer_ref.at[pl.ds(r0, RPT)], agg_sh.at[pl.ds(r0, RPT)])
    pltpu.sync_copy(zer_ref.at[pl.ds(r0, RPT)], gsum_sh.at[pl.ds(r0, RPT)])
    plsc.subcore_barrier()

    base = s * EPT

    def chunk(i, carry):
        e0 = base + i * CK
        pltpu.sync_copy(src_ref.at[pl.ds(e0, CK)], idx_s)
        pltpu.sync_copy(dst_ref.at[pl.ds(e0, CK)], idx_d)
        for j in range(CK // 16):
            sl = pl.ds(j * 16, 16)
            idx_g[sl] = idx_s[sl] + cn
        ga = pltpu.async_copy(hav_ref.at[idx_g], buf_av, sem)
        for j in range(CK // 16):
            sl = pl.ds(j * 16, 16)
            idx_b[sl] = idx_d[sl] + cn
        gb = pltpu.async_copy(hb2_ref.at[idx_b], buf_b, sem)
        pltpu.sync_copy(etc_ref.at[pl.ds(c * E + e0, CK)], buf_e)
        ga.wait()
        gb.wait()

        def comp(t, cc):
            row = t // 4
            g = (t % 4) * 16
            a = buf_av[row, pl.ds(g, 16)]
            v = buf_av[row, pl.ds(g + H, 16)]
            b = buf_b[row, pl.ds(g, 16)]
            et = buf_e[row, pl.ds(g, 16)]
            en = a + b + et
            buf_e[row, pl.ds(g, 16)] = jnp.maximum(en, 0.0)
            gt = 1.0 / (1.0 + jnp.exp(-en))
            buf_g[row, pl.ds(g, 16)] = gt
            buf_m[row, pl.ds(g, 16)] = gt * v
            return cc

        lax.fori_loop(0, CK * 4, comp, 0)

        pltpu.sync_copy(buf_e, r_ref.at[pl.ds(c * E + e0, CK)])
        pltpu.sync_copy(buf_m, agg_sh.at[idx_d], add=True)
        pltpu.sync_copy(buf_g, gsum_sh.at[idx_d], add=True)
        return carry

    lax.fori_loop(0, EPT // CK, chunk, 0)

    plsc.subcore_barrier()
    pltpu.sync_copy(agg_sh.at[pl.ds(r0, RPT)], agg_ref.at[pl.ds(c * NP + r0, RPT)])
    pltpu.sync_copy(gsum_sh.at[pl.ds(r0, RPT)], gsum_ref.at[pl.ds(c * NP + r0, RPT)])


def _edge_phase(src, dst, hav, hb2, etc, zer):
    fn = pl.kernel(
        _edge_body,
        mesh=_mesh,
        out_type=[
            jax.ShapeDtypeStruct((2 * E, H), F32),
            jax.ShapeDtypeStruct((2 * NP, H), F32),
            jax.ShapeDtypeStruct((2 * NP, H), F32),
        ],
        compiler_params=pltpu.CompilerParams(use_tc_tiling_on_sc=False, needs_layout_passes=False),
        scratch_types=[
            pltpu.VMEM((CK,), jnp.int32),
            pltpu.VMEM((CK,), jnp.int32),
            pltpu.VMEM((CK,), jnp.int32),
            pltpu.VMEM((CK,), jnp.int32),
            pltpu.VMEM((CK, 2 * H), F32),
            pltpu.VMEM((CK, H), F32),
            pltpu.VMEM((CK, H), F32),
            pltpu.VMEM((CK, H), F32),
            pltpu.VMEM((CK, H), F32),
            pltpu.VMEM_SHARED((NP, H), F32),
            pltpu.VMEM_SHARED((NP, H), F32),
            pltpu.SemaphoreType.DMA,
        ],
    )
    return fn(src, dst, hav, hb2, etc, zer)


def _pred_body(src_ref, dst_ref, hs_ref, hd_ref, etp_ref, w_ref,
               out_ref, idx_s, idx_d, buf_s, buf_d, buf_e, w_v, out_v, sem):
    c = lax.axis_index("c")
    s = lax.axis_index("s")
    wid = s * NC + c
    base = wid * EPW
    pltpu.sync_copy(w_ref, w_v)

    def chunk(i, carry):
        e0 = base + i * CK
        pltpu.sync_copy(src_ref.at[pl.ds(e0, CK)], idx_s)
        pltpu.sync_copy(dst_ref.at[pl.ds(e0, CK)], idx_d)
        ga = pltpu.async_copy(hs_ref.at[idx_s], buf_s, sem)
        gb = pltpu.async_copy(hd_ref.at[idx_d], buf_d, sem)
        pltpu.sync_copy(etp_ref.at[pl.ds(e0, CK)], buf_e)
        ga.wait()
        gb.wait()

        lanes = lax.iota(jnp.int32, 16)

        def comp(g, cc):
            vec = jnp.zeros((16,), F32)
            for r in range(16):
                row = g * 16 + r
                acc = jnp.zeros((16,), F32)
                for j in range(4):
                    sl = pl.ds(j * 16, 16)
                    z = jnp.maximum(
                        buf_s[row, sl] + buf_d[row, sl] + buf_e[row, sl], 0.0)
                    acc = acc + z * w_v[sl]
                vec = jnp.where(lanes == r, jnp.sum(acc), vec)
            out_v[pl.ds(g * 16, 16)] = vec
            return cc

        lax.fori_loop(0, CK // 16, comp, 0)
        pltpu.sync_copy(out_v, out_ref.at[pl.ds(e0, CK)])
        return carry

    lax.fori_loop(0, EPW // CK, chunk, 0)


def _pred_phase(src, dst, hs1, hd1, etp, w2):
    fn = pl.kernel(
        _pred_body,
        mesh=_mesh,
        out_type=jax.ShapeDtypeStruct((E,), F32),
        compiler_params=pltpu.CompilerParams(use_tc_tiling_on_sc=False, needs_layout_passes=False),
        scratch_types=[
            pltpu.VMEM((CK,), jnp.int32),
            pltpu.VMEM((CK,), jnp.int32),
            pltpu.VMEM((CK, H), F32),
            pltpu.VMEM((CK, H), F32),
            pltpu.VMEM((CK, H), F32),
            pltpu.VMEM((H,), F32),
            pltpu.VMEM((CK,), F32),
            pltpu.SemaphoreType.DMA,
        ],
    )
    return fn(src, dst, hs1, hd1, etp, w2)


def kernel(x, e, pe, edge_index, W_pe, b_pe, W1e, b1e, W2e, b2e,
           A, B, C, U, V, Wp1, bp1, Wp2, bp2):
    del x
    src = edge_index[0]
    dst = edge_index[1]
    zer_np = jnp.zeros((NP, H), F32)
    zer_n = jnp.zeros((N, D), F32)

    bn = 2000
    nb_n = N // bn
    bk = 2000
    nb_e = E // bk

    h = pl.pallas_call(
        _enc_h_body,
        grid=(nb_n,),
        in_specs=[
            pl.BlockSpec((bn, 18), lambda i: (i, 0)),
            _full((18, D)),
            _full((1, D)),
        ],
        out_specs=pl.BlockSpec((bn, D), lambda i: (i, 0)),
        out_shape=jax.ShapeDtypeStruct((N, D), F32),
    )(pe, W_pe, b_pe.reshape(1, D))

    et2, etc = pl.pallas_call(
        _enc_et_body,
        grid=(nb_e,),
        in_specs=[
            pl.BlockSpec((bk, 16), lambda i: (i, 0)),
            _full((16, 64)),
            _full((1, 64)),
            _full((64, D)),
            _full((1, D)),
            _full((D, D)),
        ],
        out_specs=[
            pl.BlockSpec((bk, D), lambda i: (i, 0)),
            pl.BlockSpec((2, bk, H), lambda i: (0, i, 0)),
        ],
        out_shape=[
            jax.ShapeDtypeStruct((E, D), F32),
            jax.ShapeDtypeStruct((2, E, H), F32),
        ],
    )(e, W1e, b1e.reshape(1, 64), W2e, b2e.reshape(1, D), C[0])

    etl = zer_n
    for l in range(L):
        if l > 0:
            et2, etc = pl.pallas_call(
                _etc_body,
                grid=(nb_e,),
                in_specs=[
                    pl.BlockSpec((bk, D), lambda i: (i, 0)),
                    pl.BlockSpec((2, bk, H), lambda i: (0, i, 0)),
                    _full((D, D)),
                ],
                out_specs=[
                    pl.BlockSpec((bk, D), lambda i: (i, 0)),
                    pl.BlockSpec((2, bk, H), lambda i: (0, i, 0)),
                ],
                out_shape=[
                    jax.ShapeDtypeStruct((E, D), F32),
                    jax.ShapeDtypeStruct((2, E, H), F32),
                ],
            )(et2, r_split, C[l])

        hav, hb2, hu, gl, aggl, etl = pl.pallas_call(
            _tables_body,
            grid=(nb_n,),
            in_specs=[
                pl.BlockSpec((bn, D), lambda i: (i, 0)),
                pl.BlockSpec((bn, D), lambda i: (i, 0)),
                _full((D, D)), _full((D, D)), _full((D, D)),
                _full((D, D)), _full((D, D)),
            ],
            out_specs=[
                pl.BlockSpec((2, bn, D), lambda i: (0, i, 0)),
                pl.BlockSpec((2, bn, H), lambda i: (0, i, 0)),
                pl.BlockSpec((bn, D), lambda i: (i, 0)),
                pl.BlockSpec((bn, D), lambda i: (i, 0)),
                pl.BlockSpec((bn, D), lambda i: (i, 0)),
                pl.BlockSpec((bn, D), lambda i: (i, 0)),
            ],
            out_shape=[
                jax.ShapeDtypeStruct((2, N, D), F32),
                jax.ShapeDtypeStruct((2, N, H), F32),
                jax.ShapeDtypeStruct((N, D), F32),
                jax.ShapeDtypeStruct((N, D), F32),
                jax.ShapeDtypeStruct((N, D), F32),
                jax.ShapeDtypeStruct((N, D), F32),
            ],
        )(h, etl, A[l], B[l], V[l], U[l], C[l])

        r_flat, agg_flat, gsum_flat = _edge_phase(
            src, dst,
            hav.reshape(2 * N, D),
            hb2.reshape(2 * N, H),
            etc.reshape(2 * E, H),
            zer_np,
        )
        r_split = r_flat.reshape(2, E, H)
        agg2 = agg_flat.reshape(2, NP, H)
        gsum2 = gsum_flat.reshape(2, NP, H)

        h = pl.pallas_call(
            _hupd_body,
            grid=(nb_n,),
            in_specs=[
                pl.BlockSpec((bn, D), lambda i: (i, 0)),
                pl.BlockSpec((bn, D), lambda i: (i, 0)),
                pl.BlockSpec((bn, D), lambda i: (i, 0)),
                pl.BlockSpec((bn, D), lambda i: (i, 0)),
                pl.BlockSpec((2, bn, H), lambda i: (0, i, 0)),
                pl.BlockSpec((2, bn, H), lambda i: (0, i, 0)),
            ],
            out_specs=pl.BlockSpec((bn, D), lambda i: (i, 0)),
            out_shape=jax.ShapeDtypeStruct((N, D), F32),
        )(h, hu, aggl, gl, agg2, gsum2)

    hs1, hd1 = pl.pallas_call(
        _prednode_body,
        grid=(nb_n,),
        in_specs=[
            pl.BlockSpec((bn, D), lambda i: (i, 0)),
            _full((D, H)),
            _full((D, H)),
        ],
        out_specs=[
            pl.BlockSpec((bn, H), lambda i: (i, 0)),
            pl.BlockSpec((bn, H), lambda i: (i, 0)),
        ],
        out_shape=[
            jax.ShapeDtypeStruct((N, H), F32),
            jax.ShapeDtypeStruct((N, H), F32),
        ],
    )(h, Wp1[:D], Wp1[D:2 * D])

    etp = pl.pallas_call(
        _etp_body,
        grid=(nb_e,),
        in_specs=[
            pl.BlockSpec((bk, D), lambda i: (i, 0)),
            pl.BlockSpec((2, bk, H), lambda i: (0, i, 0)),
            _full((D, H)),
            _full((1, H)),
        ],
        out_specs=pl.BlockSpec((bk, H), lambda i: (i, 0)),
        out_shape=jax.ShapeDtypeStruct((E, H), F32),
    )(et2, r_split, Wp1[2 * D:], bp1.reshape(1, H))

    scores = _pred_phase(src, dst, hs1, hd1, etp, Wp2.reshape(H))
    return scores.reshape(E, 1) + bp2


# speedup vs baseline: 1.5593x; 1.0919x over previous
import jax
import jax.numpy as jnp
from jax import lax
from jax.experimental import pallas as pl
from jax.experimental.pallas import tpu as pltpu
from jax.experimental.pallas import tpu_sc as plsc

F32 = jnp.float32

N = 10000
E = 320000
D = 128
H = 64
NP = 10240
L = 3

NS = 16
NC = 2
CK = 40
EPT = E // NS
EPW = E // (NS * NC)
RPT = NP // NS
NCH = EPT // CK
ECH = E // CK
NCHP = EPW // CK
GE = 20
GP = 50
PCK = 48

_mesh = plsc.VectorSubcoreMesh(core_axis_name="c", subcore_axis_name="s")

_SC_PARAMS = pltpu.CompilerParams(
    use_tc_tiling_on_sc=False, needs_layout_passes=False)


def _sigmoid(x):
    return 1.0 / (1.0 + jnp.exp(-x))


def _enc_h_body(pe_ref, wpe_ref, bpe_ref, h_ref):
    h_ref[...] = (
        jnp.dot(pe_ref[...], wpe_ref[...], preferred_element_type=F32)
        + bpe_ref[...]
    )


def _enc_et_body(e_ref, w1_ref, b1_ref, w2_ref, b2_ref, c0_ref, et_ref, etc_ref):
    t = jnp.maximum(
        jnp.dot(e_ref[...], w1_ref[...], preferred_element_type=F32) + b1_ref[...],
        0.0,
    )
    et = jnp.dot(t, w2_ref[...], preferred_element_type=F32) + b2_ref[...]
    et_ref[...] = et
    etc = jnp.dot(et, c0_ref[...], preferred_element_type=F32)
    etc_ref[0] = etc[:, :H]
    etc_ref[1] = etc[:, H:]


def _tables_body(h_ref, etl_ref, a_ref, b_ref, v_ref, u_ref, c_ref,
                 hav_ref, hb2_ref, hu_ref, gl_ref, aggl_ref, etln_ref):
    h = h_ref[...]
    hA = jnp.dot(h, a_ref[...], preferred_element_type=F32)
    hB = jnp.dot(h, b_ref[...], preferred_element_type=F32)
    hV = jnp.dot(h, v_ref[...], preferred_element_type=F32)
    hU = jnp.dot(h, u_ref[...], preferred_element_type=F32)
    eL = hA + hB + jnp.dot(etl_ref[...], c_ref[...], preferred_element_type=F32)
    gL = _sigmoid(eL)
    hav_ref[0] = jnp.concatenate([hA[:, :H], hV[:, :H]], axis=1)
    hav_ref[1] = jnp.concatenate([hA[:, H:], hV[:, H:]], axis=1)
    hb2_ref[0] = hB[:, :H]
    hb2_ref[1] = hB[:, H:]
    hu_ref[...] = hU
    gl_ref[...] = gL
    aggl_ref[...] = gL * hV
    etln_ref[...] = etl_ref[...] + jnp.maximum(eL, 0.0)


def _etc_body(et2_ref, r_ref, c_ref, et2n_ref, etc_ref):
    rfull = jnp.concatenate([r_ref[0], r_ref[1]], axis=1)
    etn = et2_ref[...] + rfull
    et2n_ref[...] = etn
    etc = jnp.dot(etn, c_ref[...], preferred_element_type=F32)
    etc_ref[0] = etc[:, :H]
    etc_ref[1] = etc[:, H:]


def _etp_body(et2_ref, r_ref, w_ref, b_ref, etp_ref):
    etn = et2_ref[...] + jnp.concatenate([r_ref[0], r_ref[1]], axis=1)
    etp_ref[...] = jnp.dot(etn, w_ref[...], preferred_element_type=F32) + b_ref[...]


def _hupd_body(h_ref, hu_ref, aggl_ref, gl_ref, acc_ref, hn_ref):
    agg = jnp.concatenate([acc_ref[0][:, :H], acc_ref[1][:, :H]], axis=1)
    gs = jnp.concatenate([acc_ref[0][:, H:], acc_ref[1][:, H:]], axis=1)
    agg = agg + aggl_ref[...]
    gs = gs + gl_ref[...] + 1e-6
    hn_ref[...] = h_ref[...] + jnp.maximum(hu_ref[...] + agg / gs, 0.0)


def _prednode_body(h_ref, wa_ref, wb_ref, hs_ref, hd_ref):
    h = h_ref[...]
    hs_ref[...] = jnp.dot(h, wa_ref[...], preferred_element_type=F32)
    hd_ref[...] = jnp.dot(h, wb_ref[...], preferred_element_type=F32)


def _full(shape):
    return pl.BlockSpec(shape, lambda i: tuple(0 for _ in shape))


def _edge_body(srcg_ref, dstg_ref, dstl_ref, hav_ref, hb2_ref, etc_ref, zer_ref,
               r_ref, acc_ref,
               sg2, dg2, dl0, dl1,
               av0, b0, e0, mg0,
               av1, b1, e1, mg1,
               acc_sh, sgs0, sgs1, sws0, sws1):
    c = lax.axis_index("c")
    s = lax.axis_index("s")
    r0 = s * RPT

    pltpu.sync_copy(zer_ref.at[pl.ds(r0, RPT)], acc_sh.at[pl.ds(r0, RPT)])
    plsc.subcore_barrier()

    bufs = ((av0, b0, e0, mg0, dl0, sgs0, sws0),
            (av1, b1, e1, mg1, dl1, sgs1, sws1))

    def gathers(ch, gi, p):
        av, bb, eb, _, dl, sg, _ = bufs[p]
        ebase = c * E + s * EPT + ch * CK
        return (pltpu.async_copy(hav_ref.at[sg2.at[gi]], av, sg),
                pltpu.async_copy(hb2_ref.at[dg2.at[gi]], bb, sg),
                pltpu.async_copy(etc_ref.at[pl.ds(ebase, CK)], eb, sg),
                pltpu.async_copy(dstl_ref.at[s * NCH + ch], dl, sg))

    def compute(p):
        av, bb, eb, mg, _, _, _ = bufs[p]

        def rowfn(row):
            for j in range(4):
                sl = pl.ds(j * 16, 16)
                slv = pl.ds(H + j * 16, 16)
                en = av[row, sl] + bb[row, sl] + eb[row, sl]
                eb[row, sl] = jnp.maximum(en, 0.0)
                gt = 1.0 / (1.0 + jnp.exp(-en))
                mg[row, sl] = gt * av[row, slv]
                mg[row, slv] = gt

        def rowfn_i(row, cc):
            rowfn(row)
            return cc

        lax.fori_loop(0, CK, rowfn_i, 0)

    def writes(ch, gi, p):
        _, _, eb, mg, dl, _, sw = bufs[p]
        ebase = c * E + s * EPT + ch * CK
        w = pltpu.async_copy(eb, r_ref.at[pl.ds(ebase, CK)], sw)
        pltpu.sync_copy(mg, acc_sh.at[dl], add=True)
        return (w,)

    def group(g, cc):
        gbase = s * NCH + g * GE
        pltpu.sync_copy(srcg_ref.at[pl.ds(c * ECH + gbase, GE)], sg2)
        pltpu.sync_copy(dstg_ref.at[pl.ds(c * ECH + gbase, GE)], dg2)

        def body(i, cc2):
            gi0 = 2 * i
            gi1 = 2 * i + 1
            ch0 = g * GE + gi0
            ch1 = g * GE + gi1
            h0 = gathers(ch0, gi0, 0)
            h1 = gathers(ch1, gi1, 1)
            for hh in h0:
                hh.wait()
            compute(0)
            w0 = writes(ch0, gi0, 0)
            for hh in h1:
                hh.wait()
            compute(1)
            w1 = writes(ch1, gi1, 1)
            for ww in w0:
                ww.wait()
            for ww in w1:
                ww.wait()
            return cc2

        lax.fori_loop(0, GE // 2, body, 0)
        return cc

    lax.fori_loop(0, NCH // GE, group, 0)

    plsc.subcore_barrier()
    pltpu.sync_copy(acc_sh.at[pl.ds(r0, RPT)], acc_ref.at[pl.ds(c * NP + r0, RPT)])


def _edge_phase(srcg, dstg, dstl, hav, hb2, etc, zer):
    fn = pl.kernel(
        _edge_body,
        mesh=_mesh,
        out_type=[
            jax.ShapeDtypeStruct((2 * E, H), F32),
            jax.ShapeDtypeStruct((2 * NP, 2 * H), F32),
        ],
        compiler_params=_SC_PARAMS,
        scratch_types=[
            pltpu.VMEM((GE, CK), jnp.int32),
            pltpu.VMEM((GE, CK), jnp.int32),
            pltpu.VMEM((CK,), jnp.int32),
            pltpu.VMEM((CK,), jnp.int32),
            pltpu.VMEM((CK, 2 * H), F32),
            pltpu.VMEM((CK, H), F32),
            pltpu.VMEM((CK, H), F32),
            pltpu.VMEM((CK, 2 * H), F32),
            pltpu.VMEM((CK, 2 * H), F32),
            pltpu.VMEM((CK, H), F32),
            pltpu.VMEM((CK, H), F32),
            pltpu.VMEM((CK, 2 * H), F32),
            pltpu.VMEM_SHARED((NP, 2 * H), F32),
            pltpu.SemaphoreType.DMA,
            pltpu.SemaphoreType.DMA,
            pltpu.SemaphoreType.DMA,
            pltpu.SemaphoreType.DMA,
        ],
    )
    return fn(srcg, dstg, dstl, hav, hb2, etc, zer)


def _pred_body(srcl_ref, dstl_ref, hs_ref, hd_ref, etp_ref, w_ref,
               out_ref,
               sl2, dl2, w_v,
               bs0, bd0, be0, ov0,
               bs1, bd1, be1, ov1,
               sgs0, sgs1, sws0, sws1):
    c = lax.axis_index("c")
    s = lax.axis_index("s")
    wid = s * NC + c
    base = wid * EPW
    pltpu.sync_copy(w_ref, w_v)

    bufs = ((bs0, bd0, be0, ov0, sgs0, sws0),
            (bs1, bd1, be1, ov1, sgs1, sws1))

    def gathers(ch, gi, p):
        bs, bd, be, _, sg, _ = bufs[p]
        return (pltpu.async_copy(hs_ref.at[sl2.at[gi]], bs.at[pl.ds(0, CK)], sg),
                pltpu.async_copy(hd_ref.at[dl2.at[gi]], bd.at[pl.ds(0, CK)], sg),
                pltpu.async_copy(etp_ref.at[pl.ds(base + ch * CK, CK)],
                                 be.at[pl.ds(0, CK)], sg))

    lanes = lax.iota(jnp.int32, 16)

    def compute(p):
        bs, bd, be, ov, _, _ = bufs[p]

        def gfn(g):
            vec = jnp.zeros((16,), F32)
            for r in range(16):
                row = g * 16 + r
                acc = jnp.zeros((16,), F32)
                for j in range(4):
                    sl = pl.ds(j * 16, 16)
                    z = jnp.maximum(bs[row, sl] + bd[row, sl] + be[row, sl], 0.0)
                    acc = acc + z * w_v[sl]
                vec = jnp.where(lanes == r, jnp.sum(acc), vec)
            ov[pl.ds(g * 16, 16)] = vec

        def gfn_i(g, cc):
            gfn(g)
            return cc

        lax.fori_loop(0, PCK // 16, gfn_i, 0)

    def group(g, cc):
        gbase = wid * NCHP + g * GP
        pltpu.sync_copy(srcl_ref.at[pl.ds(gbase, GP)], sl2)
        pltpu.sync_copy(dstl_ref.at[pl.ds(gbase, GP)], dl2)

        def body(i, cc2):
            gi0 = 2 * i
            gi1 = 2 * i + 1
            ch0 = g * GP + gi0
            ch1 = g * GP + gi1
            h0 = gathers(ch0, gi0, 0)
            h1 = gathers(ch1, gi1, 1)
            for hh in h0:
                hh.wait()
            compute(0)
            w0 = pltpu.async_copy(ov0.at[pl.ds(0, CK)],
                                  out_ref.at[pl.ds(base + ch0 * CK, CK)],
                                  bufs[0][5])
            for hh in h1:
                hh.wait()
            compute(1)
            w1 = pltpu.async_copy(ov1.at[pl.ds(0, CK)],
                                  out_ref.at[pl.ds(base + ch1 * CK, CK)],
                                  bufs[1][5])
            w0.wait()
            w1.wait()
            return cc2

        lax.fori_loop(0, GP // 2, body, 0)
        return cc

    lax.fori_loop(0, NCHP // GP, group, 0)


def _pred_phase(srcl, dstl, hs1, hd1, etp, w2):
    fn = pl.kernel(
        _pred_body,
        mesh=_mesh,
        out_type=jax.ShapeDtypeStruct((E,), F32),
        compiler_params=_SC_PARAMS,
        scratch_types=[
            pltpu.VMEM((GP, CK), jnp.int32),
            pltpu.VMEM((GP, CK), jnp.int32),
            pltpu.VMEM((H,), F32),
            pltpu.VMEM((PCK, H), F32),
            pltpu.VMEM((PCK, H), F32),
            pltpu.VMEM((PCK, H), F32),
            pltpu.VMEM((PCK,), F32),
            pltpu.VMEM((PCK, H), F32),
            pltpu.VMEM((PCK, H), F32),
            pltpu.VMEM((PCK, H), F32),
            pltpu.VMEM((PCK,), F32),
            pltpu.SemaphoreType.DMA,
            pltpu.SemaphoreType.DMA,
            pltpu.SemaphoreType.DMA,
            pltpu.SemaphoreType.DMA,
        ],
    )
    return fn(srcl, dstl, hs1, hd1, etp, w2)


def kernel(x, e, pe, edge_index, W_pe, b_pe, W1e, b1e, W2e, b2e,
           A, B, C, U, V, Wp1, bp1, Wp2, bp2):
    del x
    src = edge_index[0]
    dst = edge_index[1]
    srcg = jnp.concatenate([src, src + N]).reshape(2 * ECH, CK)
    dstg = jnp.concatenate([dst, dst + N]).reshape(2 * ECH, CK)
    dstl = dst.reshape(ECH, CK)
    srcl = src.reshape(ECH, CK)
    zer_np = jnp.zeros((NP, 2 * H), F32)
    zer_n = jnp.zeros((N, D), F32)

    bn = 2000
    nb_n = N // bn
    bk = 2000
    nb_e = E // bk

    h = pl.pallas_call(
        _enc_h_body,
        grid=(nb_n,),
        in_specs=[
            pl.BlockSpec((bn, 18), lambda i: (i, 0)),
            _full((18, D)),
            _full((1, D)),
        ],
        out_specs=pl.BlockSpec((bn, D), lambda i: (i, 0)),
        out_shape=jax.ShapeDtypeStruct((N, D), F32),
    )(pe, W_pe, b_pe.reshape(1, D))

    et2, etc = pl.pallas_call(
        _enc_et_body,
        grid=(nb_e,),
        in_specs=[
            pl.BlockSpec((bk, 16), lambda i: (i, 0)),
            _full((16, 64)),
            _full((1, 64)),
            _full((64, D)),
            _full((1, D)),
            _full((D, D)),
        ],
        out_specs=[
            pl.BlockSpec((bk, D), lambda i: (i, 0)),
            pl.BlockSpec((2, bk, H), lambda i: (0, i, 0)),
        ],
        out_shape=[
            jax.ShapeDtypeStruct((E, D), F32),
            jax.ShapeDtypeStruct((2, E, H), F32),
        ],
    )(e, W1e, b1e.reshape(1, 64), W2e, b2e.reshape(1, D), C[0])

    etl = zer_n
    for l in range(L):
        if l > 0:
            et2, etc = pl.pallas_call(
                _etc_body,
                grid=(nb_e,),
                in_specs=[
                    pl.BlockSpec((bk, D), lambda i: (i, 0)),
                    pl.BlockSpec((2, bk, H), lambda i: (0, i, 0)),
                    _full((D, D)),
                ],
                out_specs=[
                    pl.BlockSpec((bk, D), lambda i: (i, 0)),
                    pl.BlockSpec((2, bk, H), lambda i: (0, i, 0)),
                ],
                out_shape=[
                    jax.ShapeDtypeStruct((E, D), F32),
                    jax.ShapeDtypeStruct((2, E, H), F32),
                ],
            )(et2, r_split, C[l])

        hav, hb2, hu, gl, aggl, etl = pl.pallas_call(
            _tables_body,
            grid=(nb_n,),
            in_specs=[
                pl.BlockSpec((bn, D), lambda i: (i, 0)),
                pl.BlockSpec((bn, D), lambda i: (i, 0)),
                _full((D, D)), _full((D, D)), _full((D, D)),
                _full((D, D)), _full((D, D)),
            ],
            out_specs=[
                pl.BlockSpec((2, bn, D), lambda i: (0, i, 0)),
                pl.BlockSpec((2, bn, H), lambda i: (0, i, 0)),
                pl.BlockSpec((bn, D), lambda i: (i, 0)),
                pl.BlockSpec((bn, D), lambda i: (i, 0)),
                pl.BlockSpec((bn, D), lambda i: (i, 0)),
                pl.BlockSpec((bn, D), lambda i: (i, 0)),
            ],
            out_shape=[
                jax.ShapeDtypeStruct((2, N, D), F32),
                jax.ShapeDtypeStruct((2, N, H), F32),
                jax.ShapeDtypeStruct((N, D), F32),
                jax.ShapeDtypeStruct((N, D), F32),
                jax.ShapeDtypeStruct((N, D), F32),
                jax.ShapeDtypeStruct((N, D), F32),
            ],
        )(h, etl, A[l], B[l], V[l], U[l], C[l])

        r_flat, acc_flat = _edge_phase(
            srcg, dstg, dstl,
            hav.reshape(2 * N, D),
            hb2.reshape(2 * N, H),
            etc.reshape(2 * E, H),
            zer_np,
        )
        r_split = r_flat.reshape(2, E, H)
        acc2 = acc_flat.reshape(2, NP, 2 * H)

        h = pl.pallas_call(
            _hupd_body,
            grid=(nb_n,),
            in_specs=[
                pl.BlockSpec((bn, D), lambda i: (i, 0)),
                pl.BlockSpec((bn, D), lambda i: (i, 0)),
                pl.BlockSpec((bn, D), lambda i: (i, 0)),
                pl.BlockSpec((bn, D), lambda i: (i, 0)),
                pl.BlockSpec((2, bn, 2 * H), lambda i: (0, i, 0)),
            ],
            out_specs=pl.BlockSpec((bn, D), lambda i: (i, 0)),
            out_shape=jax.ShapeDtypeStruct((N, D), F32),
        )(h, hu, aggl, gl, acc2)

    hs1, hd1 = pl.pallas_call(
        _prednode_body,
        grid=(nb_n,),
        in_specs=[
            pl.BlockSpec((bn, D), lambda i: (i, 0)),
            _full((D, H)),
            _full((D, H)),
        ],
        out_specs=[
            pl.BlockSpec((bn, H), lambda i: (i, 0)),
            pl.BlockSpec((bn, H), lambda i: (i, 0)),
        ],
        out_shape=[
            jax.ShapeDtypeStruct((N, H), F32),
            jax.ShapeDtypeStruct((N, H), F32),
        ],
    )(h, Wp1[:D], Wp1[D:2 * D])

    etp = pl.pallas_call(
        _etp_body,
        grid=(nb_e,),
        in_specs=[
            pl.BlockSpec((bk, D), lambda i: (i, 0)),
            pl.BlockSpec((2, bk, H), lambda i: (0, i, 0)),
            _full((D, H)),
            _full((1, H)),
        ],
        out_specs=pl.BlockSpec((bk, H), lambda i: (i, 0)),
        out_shape=jax.ShapeDtypeStruct((E, H), F32),
    )(et2, r_split, Wp1[2 * D:], bp1.reshape(1, H))

    scores = _pred_phase(srcl, dstl, hs1, hd1, etp, Wp2.reshape(H))
    return scores.reshape(E, 1) + bp2


# speedup vs baseline: 2.8454x; 1.8248x over previous
import jax
import jax.numpy as jnp
from jax import lax
from jax.experimental import pallas as pl
from jax.experimental.pallas import tpu as pltpu
from jax.experimental.pallas import tpu_sc as plsc

F32 = jnp.float32

N = 10000
E = 320000
D = 128
H = 64
NP = 10240
L = 3

NS = 16
NC = 2
CK = 40
EPT = E // NS
EPW = E // (NS * NC)
RPT = NP // NS
NCH = EPT // CK
ECH = E // CK
NCHP = EPW // CK
GE = 20
GP = 50
PCK = 48

_mesh = plsc.VectorSubcoreMesh(core_axis_name="c", subcore_axis_name="s")

_SC_PARAMS = pltpu.CompilerParams(
    use_tc_tiling_on_sc=False, needs_layout_passes=False)


def _sigmoid(x):
    return 1.0 / (1.0 + jnp.exp(-x))


def _enc_h_body(pe_ref, wpe_ref, bpe_ref, h_ref):
    h_ref[...] = (
        jnp.dot(pe_ref[...], wpe_ref[...], preferred_element_type=F32)
        + bpe_ref[...]
    )


def _enc_et_body(e_ref, w1_ref, b1_ref, w2_ref, b2_ref, c0_ref, et_ref, etc_ref):
    t = jnp.maximum(
        jnp.dot(e_ref[...], w1_ref[...], preferred_element_type=F32) + b1_ref[...],
        0.0,
    )
    et = jnp.dot(t, w2_ref[...], preferred_element_type=F32) + b2_ref[...]
    et_ref[...] = et
    etc = jnp.dot(et, c0_ref[...], preferred_element_type=F32)
    etc_ref[0] = etc[:, :H]
    etc_ref[1] = etc[:, H:]


def _tables_body(h_ref, etl_ref, a_ref, b_ref, v_ref, u_ref, c_ref,
                 hav_ref, hb2_ref, hu_ref, gl_ref, aggl_ref, etln_ref):
    h = h_ref[...]
    hA = jnp.dot(h, a_ref[...], preferred_element_type=F32)
    hB = jnp.dot(h, b_ref[...], preferred_element_type=F32)
    hV = jnp.dot(h, v_ref[...], preferred_element_type=F32)
    hU = jnp.dot(h, u_ref[...], preferred_element_type=F32)
    eL = hA + hB + jnp.dot(etl_ref[...], c_ref[...], preferred_element_type=F32)
    gL = _sigmoid(eL)
    hav_ref[0] = jnp.concatenate([hA[:, :H], hV[:, :H]], axis=1)
    hav_ref[1] = jnp.concatenate([hA[:, H:], hV[:, H:]], axis=1)
    hb2_ref[0] = hB[:, :H]
    hb2_ref[1] = hB[:, H:]
    hu_ref[...] = hU
    gl_ref[...] = gL
    aggl_ref[...] = gL * hV
    etln_ref[...] = etl_ref[...] + jnp.maximum(eL, 0.0)


def _etc_body(et2_ref, r_ref, c_ref, et2n_ref, etc_ref):
    rfull = jnp.concatenate([r_ref[0], r_ref[1]], axis=1)
    etn = et2_ref[...] + rfull
    et2n_ref[...] = etn
    etc = jnp.dot(etn, c_ref[...], preferred_element_type=F32)
    etc_ref[0] = etc[:, :H]
    etc_ref[1] = etc[:, H:]


def _etp_body(et2_ref, r_ref, w_ref, b_ref, etp_ref):
    etn = et2_ref[...] + jnp.concatenate([r_ref[0], r_ref[1]], axis=1)
    etp_ref[...] = jnp.dot(etn, w_ref[...], preferred_element_type=F32) + b_ref[...]


def _hupd_body(h_ref, hu_ref, aggl_ref, gl_ref, acc_ref, hn_ref):
    agg = jnp.concatenate([acc_ref[0][:, :H], acc_ref[1][:, :H]], axis=1)
    gs = jnp.concatenate([acc_ref[0][:, H:], acc_ref[1][:, H:]], axis=1)
    agg = agg + aggl_ref[...]
    gs = gs + gl_ref[...] + 1e-6
    hn_ref[...] = h_ref[...] + jnp.maximum(hu_ref[...] + agg / gs, 0.0)


def _prednode_body(h_ref, wa_ref, wb_ref, hs_ref, hd_ref):
    h = h_ref[...]
    hs_ref[...] = jnp.dot(h, wa_ref[...], preferred_element_type=F32)
    hd_ref[...] = jnp.dot(h, wb_ref[...], preferred_element_type=F32)


def _full(shape):
    return pl.BlockSpec(shape, lambda i: tuple(0 for _ in shape))


def _edge_body(srcg_ref, dstg_ref, dstl_ref, hav_ref, hb2_ref, etc_ref, zer_ref,
               r_ref, acc_ref,
               sg2, dg2, dl0, dl1,
               av0, b0, e0, mg0,
               av1, b1, e1, mg1,
               acc_sh, sgs0, sgs1, sws0, sws1):
    c = lax.axis_index("c")
    s = lax.axis_index("s")
    r0 = s * RPT

    pltpu.sync_copy(zer_ref.at[pl.ds(r0, RPT)], acc_sh.at[pl.ds(r0, RPT)])
    plsc.subcore_barrier()

    bufs = ((av0, b0, e0, mg0, dl0, sgs0, sws0),
            (av1, b1, e1, mg1, dl1, sgs1, sws1))

    def gathers(ch, gi, p):
        av, bb, eb, _, dl, sg, _ = bufs[p]
        ebase = c * E + s * EPT + ch * CK
        return (pltpu.async_copy(hav_ref.at[sg2.at[gi]], av, sg),
                pltpu.async_copy(hb2_ref.at[dg2.at[gi]], bb, sg),
                pltpu.async_copy(etc_ref.at[pl.ds(ebase, CK)], eb, sg),
                pltpu.async_copy(dstl_ref.at[s * NCH + ch], dl, sg))

    def compute(p):
        av, bb, eb, mg, _, _, _ = bufs[p]

        def rowfn(row):
            for j in range(4):
                sl = pl.ds(j * 16, 16)
                slv = pl.ds(H + j * 16, 16)
                en = av[row, sl] + bb[row, sl] + eb[row, sl]
                eb[row, sl] = jnp.maximum(en, 0.0)
                gt = 1.0 / (1.0 + jnp.exp(-en))
                mg[row, sl] = gt * av[row, slv]
                mg[row, slv] = gt

        plsc.parallel_loop(0, CK, unroll=2)(rowfn)

    def writes(ch, gi, p):
        _, _, eb, mg, dl, _, sw = bufs[p]
        ebase = c * E + s * EPT + ch * CK
        w = pltpu.async_copy(eb, r_ref.at[pl.ds(ebase, CK)], sw)
        pltpu.sync_copy(mg, acc_sh.at[dl], add=True)
        return (w,)

    def group(g, cc):
        gbase = s * NCH + g * GE
        pltpu.sync_copy(srcg_ref.at[pl.ds(c * ECH + gbase, GE)], sg2)
        pltpu.sync_copy(dstg_ref.at[pl.ds(c * ECH + gbase, GE)], dg2)

        def body(i, cc2):
            gi0 = 2 * i
            gi1 = 2 * i + 1
            ch0 = g * GE + gi0
            ch1 = g * GE + gi1
            h0 = gathers(ch0, gi0, 0)
            h1 = gathers(ch1, gi1, 1)
            for hh in h0:
                hh.wait()
            compute(0)
            w0 = writes(ch0, gi0, 0)
            for hh in h1:
                hh.wait()
            compute(1)
            w1 = writes(ch1, gi1, 1)
            for ww in w0:
                ww.wait()
            for ww in w1:
                ww.wait()
            return cc2

        lax.fori_loop(0, GE // 2, body, 0)
        return cc

    lax.fori_loop(0, NCH // GE, group, 0)

    plsc.subcore_barrier()
    pltpu.sync_copy(acc_sh.at[pl.ds(r0, RPT)], acc_ref.at[pl.ds(c * NP + r0, RPT)])


def _edge_phase(srcg, dstg, dstl, hav, hb2, etc, zer):
    fn = pl.kernel(
        _edge_body,
        mesh=_mesh,
        out_type=[
            jax.ShapeDtypeStruct((2 * E, H), F32),
            jax.ShapeDtypeStruct((2 * NP, 2 * H), F32),
        ],
        compiler_params=_SC_PARAMS,
        scratch_types=[
            pltpu.VMEM((GE, CK), jnp.int32),
            pltpu.VMEM((GE, CK), jnp.int32),
            pltpu.VMEM((CK,), jnp.int32),
            pltpu.VMEM((CK,), jnp.int32),
            pltpu.VMEM((CK, 2 * H), F32),
            pltpu.VMEM((CK, H), F32),
            pltpu.VMEM((CK, H), F32),
            pltpu.VMEM((CK, 2 * H), F32),
            pltpu.VMEM((CK, 2 * H), F32),
            pltpu.VMEM((CK, H), F32),
            pltpu.VMEM((CK, H), F32),
            pltpu.VMEM((CK, 2 * H), F32),
            pltpu.VMEM_SHARED((NP, 2 * H), F32),
            pltpu.SemaphoreType.DMA,
            pltpu.SemaphoreType.DMA,
            pltpu.SemaphoreType.DMA,
            pltpu.SemaphoreType.DMA,
        ],
    )
    return fn(srcg, dstg, dstl, hav, hb2, etc, zer)


def _pred_body(srcl_ref, dstl_ref, hs_ref, hd_ref, etp_ref, w_ref,
               out_ref,
               sl2, dl2, w_v,
               bs0, bd0, be0, ov0,
               bs1, bd1, be1, ov1,
               sgs0, sgs1, sws0, sws1):
    c = lax.axis_index("c")
    s = lax.axis_index("s")
    wid = s * NC + c
    base = wid * EPW
    pltpu.sync_copy(w_ref, w_v)

    bufs = ((bs0, bd0, be0, ov0, sgs0, sws0),
            (bs1, bd1, be1, ov1, sgs1, sws1))

    def gathers(ch, gi, p):
        bs, bd, be, _, sg, _ = bufs[p]
        return (pltpu.async_copy(hs_ref.at[sl2.at[gi]], bs.at[pl.ds(0, CK)], sg),
                pltpu.async_copy(hd_ref.at[dl2.at[gi]], bd.at[pl.ds(0, CK)], sg),
                pltpu.async_copy(etp_ref.at[pl.ds(base + ch * CK, CK)],
                                 be.at[pl.ds(0, CK)], sg))

    lanes = lax.iota(jnp.int32, 16)

    def compute(p):
        bs, bd, be, ov, _, _ = bufs[p]

        def gfn(g):
            vec = jnp.zeros((16,), F32)
            for r in range(16):
                row = g * 16 + r
                acc = jnp.zeros((16,), F32)
                for j in range(4):
                    sl = pl.ds(j * 16, 16)
                    z = jnp.maximum(bs[row, sl] + bd[row, sl] + be[row, sl], 0.0)
                    acc = acc + z * w_v[sl]
                vec = jnp.where(lanes == r, jnp.sum(acc), vec)
            ov[pl.ds(g * 16, 16)] = vec

        def gfn_i(g, cc):
            gfn(g)
            return cc

        lax.fori_loop(0, PCK // 16, gfn_i, 0)

    def group(g, cc):
        gbase = wid * NCHP + g * GP
        pltpu.sync_copy(srcl_ref.at[pl.ds(gbase, GP)], sl2)
        pltpu.sync_copy(dstl_ref.at[pl.ds(gbase, GP)], dl2)

        def body(i, cc2):
            gi0 = 2 * i
            gi1 = 2 * i + 1
            ch0 = g * GP + gi0
            ch1 = g * GP + gi1
            h0 = gathers(ch0, gi0, 0)
            h1 = gathers(ch1, gi1, 1)
            for hh in h0:
                hh.wait()
            compute(0)
            w0 = pltpu.async_copy(ov0.at[pl.ds(0, CK)],
                                  out_ref.at[pl.ds(base + ch0 * CK, CK)],
                                  bufs[0][5])
            for hh in h1:
                hh.wait()
            compute(1)
            w1 = pltpu.async_copy(ov1.at[pl.ds(0, CK)],
                                  out_ref.at[pl.ds(base + ch1 * CK, CK)],
                                  bufs[1][5])
            w0.wait()
            w1.wait()
            return cc2

        lax.fori_loop(0, GP // 2, body, 0)
        return cc

    lax.fori_loop(0, NCHP // GP, group, 0)


def _pred_phase(srcl, dstl, hs1, hd1, etp, w2):
    fn = pl.kernel(
        _pred_body,
        mesh=_mesh,
        out_type=jax.ShapeDtypeStruct((E,), F32),
        compiler_params=_SC_PARAMS,
        scratch_types=[
            pltpu.VMEM((GP, CK), jnp.int32),
            pltpu.VMEM((GP, CK), jnp.int32),
            pltpu.VMEM((H,), F32),
            pltpu.VMEM((PCK, H), F32),
            pltpu.VMEM((PCK, H), F32),
            pltpu.VMEM((PCK, H), F32),
            pltpu.VMEM((PCK,), F32),
            pltpu.VMEM((PCK, H), F32),
            pltpu.VMEM((PCK, H), F32),
            pltpu.VMEM((PCK, H), F32),
            pltpu.VMEM((PCK,), F32),
            pltpu.SemaphoreType.DMA,
            pltpu.SemaphoreType.DMA,
            pltpu.SemaphoreType.DMA,
            pltpu.SemaphoreType.DMA,
        ],
    )
    return fn(srcl, dstl, hs1, hd1, etp, w2)


def kernel(x, e, pe, edge_index, W_pe, b_pe, W1e, b1e, W2e, b2e,
           A, B, C, U, V, Wp1, bp1, Wp2, bp2):
    del x
    src = edge_index[0]
    dst = edge_index[1]
    srcg = jnp.concatenate([src, src + N]).reshape(2 * ECH, CK)
    dstg = jnp.concatenate([dst, dst + N]).reshape(2 * ECH, CK)
    dstl = dst.reshape(ECH, CK)
    srcl = src.reshape(ECH, CK)
    zer_np = jnp.zeros((NP, 2 * H), F32)
    zer_n = jnp.zeros((N, D), F32)

    bn = 2000
    nb_n = N // bn
    bk = 2000
    nb_e = E // bk

    h = pl.pallas_call(
        _enc_h_body,
        grid=(nb_n,),
        in_specs=[
            pl.BlockSpec((bn, 18), lambda i: (i, 0)),
            _full((18, D)),
            _full((1, D)),
        ],
        out_specs=pl.BlockSpec((bn, D), lambda i: (i, 0)),
        out_shape=jax.ShapeDtypeStruct((N, D), F32),
    )(pe, W_pe, b_pe.reshape(1, D))

    et2, etc = pl.pallas_call(
        _enc_et_body,
        grid=(nb_e,),
        in_specs=[
            pl.BlockSpec((bk, 16), lambda i: (i, 0)),
            _full((16, 64)),
            _full((1, 64)),
            _full((64, D)),
            _full((1, D)),
            _full((D, D)),
        ],
        out_specs=[
            pl.BlockSpec((bk, D), lambda i: (i, 0)),
            pl.BlockSpec((2, bk, H), lambda i: (0, i, 0)),
        ],
        out_shape=[
            jax.ShapeDtypeStruct((E, D), F32),
            jax.ShapeDtypeStruct((2, E, H), F32),
        ],
    )(e, W1e, b1e.reshape(1, 64), W2e, b2e.reshape(1, D), C[0])

    etl = zer_n
    for l in range(L):
        if l > 0:
            et2, etc = pl.pallas_call(
                _etc_body,
                grid=(nb_e,),
                in_specs=[
                    pl.BlockSpec((bk, D), lambda i: (i, 0)),
                    pl.BlockSpec((2, bk, H), lambda i: (0, i, 0)),
                    _full((D, D)),
                ],
                out_specs=[
                    pl.BlockSpec((bk, D), lambda i: (i, 0)),
                    pl.BlockSpec((2, bk, H), lambda i: (0, i, 0)),
                ],
                out_shape=[
                    jax.ShapeDtypeStruct((E, D), F32),
                    jax.ShapeDtypeStruct((2, E, H), F32),
                ],
            )(et2, r_split, C[l])

        hav, hb2, hu, gl, aggl, etl = pl.pallas_call(
            _tables_body,
            grid=(nb_n,),
            in_specs=[
                pl.BlockSpec((bn, D), lambda i: (i, 0)),
                pl.BlockSpec((bn, D), lambda i: (i, 0)),
                _full((D, D)), _full((D, D)), _full((D, D)),
                _full((D, D)), _full((D, D)),
            ],
            out_specs=[
                pl.BlockSpec((2, bn, D), lambda i: (0, i, 0)),
                pl.BlockSpec((2, bn, H), lambda i: (0, i, 0)),
                pl.BlockSpec((bn, D), lambda i: (i, 0)),
                pl.BlockSpec((bn, D), lambda i: (i, 0)),
                pl.BlockSpec((bn, D), lambda i: (i, 0)),
                pl.BlockSpec((bn, D), lambda i: (i, 0)),
            ],
            out_shape=[
                jax.ShapeDtypeStruct((2, N, D), F32),
                jax.ShapeDtypeStruct((2, N, H), F32),
                jax.ShapeDtypeStruct((N, D), F32),
                jax.ShapeDtypeStruct((N, D), F32),
                jax.ShapeDtypeStruct((N, D), F32),
                jax.ShapeDtypeStruct((N, D), F32),
            ],
        )(h, etl, A[l], B[l], V[l], U[l], C[l])

        r_flat, acc_flat = _edge_phase(
            srcg, dstg, dstl,
            hav.reshape(2 * N, D),
            hb2.reshape(2 * N, H),
            etc.reshape(2 * E, H),
            zer_np,
        )
        r_split = r_flat.reshape(2, E, H)
        acc2 = acc_flat.reshape(2, NP, 2 * H)

        h = pl.pallas_call(
            _hupd_body,
            grid=(nb_n,),
            in_specs=[
                pl.BlockSpec((bn, D), lambda i: (i, 0)),
                pl.BlockSpec((bn, D), lambda i: (i, 0)),
                pl.BlockSpec((bn, D), lambda i: (i, 0)),
                pl.BlockSpec((bn, D), lambda i: (i, 0)),
                pl.BlockSpec((2, bn, 2 * H), lambda i: (0, i, 0)),
            ],
            out_specs=pl.BlockSpec((bn, D), lambda i: (i, 0)),
            out_shape=jax.ShapeDtypeStruct((N, D), F32),
        )(h, hu, aggl, gl, acc2)

    hs1, hd1 = pl.pallas_call(
        _prednode_body,
        grid=(nb_n,),
        in_specs=[
            pl.BlockSpec((bn, D), lambda i: (i, 0)),
            _full((D, H)),
            _full((D, H)),
        ],
        out_specs=[
            pl.BlockSpec((bn, H), lambda i: (i, 0)),
            pl.BlockSpec((bn, H), lambda i: (i, 0)),
        ],
        out_shape=[
            jax.ShapeDtypeStruct((N, H), F32),
            jax.ShapeDtypeStruct((N, H), F32),
        ],
    )(h, Wp1[:D], Wp1[D:2 * D])

    etp = pl.pallas_call(
        _etp_body,
        grid=(nb_e,),
        in_specs=[
            pl.BlockSpec((bk, D), lambda i: (i, 0)),
            pl.BlockSpec((2, bk, H), lambda i: (0, i, 0)),
            _full((D, H)),
            _full((1, H)),
        ],
        out_specs=pl.BlockSpec((bk, H), lambda i: (i, 0)),
        out_shape=jax.ShapeDtypeStruct((E, H), F32),
    )(et2, r_split, Wp1[2 * D:], bp1.reshape(1, H))

    scores = _pred_phase(srcl, dstl, hs1, hd1, etp, Wp2.reshape(H))
    return scores.reshape(E, 1) + bp2


# speedup vs baseline: 2.8510x; 1.0020x over previous
import jax
import jax.numpy as jnp
from jax import lax
from jax.experimental import pallas as pl
from jax.experimental.pallas import tpu as pltpu
from jax.experimental.pallas import tpu_sc as plsc

F32 = jnp.float32

N = 10000
E = 320000
D = 128
H = 64
NP = 10240
L = 3

NS = 16
NC = 2
CK = 40
EPT = E // NS
EPW = E // (NS * NC)
RPT = NP // NS
NCH = EPT // CK
ECH = E // CK
NCHP = EPW // CK
GE = 20
GP = 50
PCK = 48

_mesh = plsc.VectorSubcoreMesh(core_axis_name="c", subcore_axis_name="s")

_SC_PARAMS = pltpu.CompilerParams(
    use_tc_tiling_on_sc=False, needs_layout_passes=False)


def _sigmoid(x):
    return 1.0 / (1.0 + jnp.exp(-x))


def _enc_h_body(pe_ref, wpe_ref, bpe_ref, h_ref):
    h_ref[...] = (
        jnp.dot(pe_ref[...], wpe_ref[...], preferred_element_type=F32)
        + bpe_ref[...]
    )


def _enc_et_body(e_ref, w1_ref, b1_ref, w2_ref, b2_ref, c0_ref, et_ref, etc_ref):
    t = jnp.maximum(
        jnp.dot(e_ref[...], w1_ref[...], preferred_element_type=F32) + b1_ref[...],
        0.0,
    )
    et = jnp.dot(t, w2_ref[...], preferred_element_type=F32) + b2_ref[...]
    et_ref[...] = et
    etc = jnp.dot(et, c0_ref[...], preferred_element_type=F32)
    etc_ref[0] = etc[:, :H]
    etc_ref[1] = etc[:, H:]


def _tables_body(h_ref, etl_ref, a_ref, b_ref, v_ref, u_ref, c_ref,
                 hav_ref, hb2_ref, hu_ref, gl_ref, aggl_ref, etln_ref):
    h = h_ref[...]
    hA = jnp.dot(h, a_ref[...], preferred_element_type=F32)
    hB = jnp.dot(h, b_ref[...], preferred_element_type=F32)
    hV = jnp.dot(h, v_ref[...], preferred_element_type=F32)
    hU = jnp.dot(h, u_ref[...], preferred_element_type=F32)
    eL = hA + hB + jnp.dot(etl_ref[...], c_ref[...], preferred_element_type=F32)
    gL = _sigmoid(eL)
    hav_ref[0] = jnp.concatenate([hA[:, :H], hV[:, :H]], axis=1)
    hav_ref[1] = jnp.concatenate([hA[:, H:], hV[:, H:]], axis=1)
    hb2_ref[0] = hB[:, :H]
    hb2_ref[1] = hB[:, H:]
    hu_ref[...] = hU
    gl_ref[...] = gL
    aggl_ref[...] = gL * hV
    etln_ref[...] = etl_ref[...] + jnp.maximum(eL, 0.0)


def _etc_body(et2_ref, r_ref, c_ref, et2n_ref, etc_ref):
    rfull = jnp.concatenate([r_ref[0], r_ref[1]], axis=1)
    etn = et2_ref[...] + rfull
    et2n_ref[...] = etn
    etc = jnp.dot(etn, c_ref[...], preferred_element_type=F32)
    etc_ref[0] = etc[:, :H]
    etc_ref[1] = etc[:, H:]


def _etp_body(et2_ref, r_ref, w_ref, b_ref, etp_ref):
    etn = et2_ref[...] + jnp.concatenate([r_ref[0], r_ref[1]], axis=1)
    etp_ref[...] = jnp.dot(etn, w_ref[...], preferred_element_type=F32) + b_ref[...]


def _hupd_body(h_ref, hu_ref, aggl_ref, gl_ref, acc_ref, hn_ref):
    agg = jnp.concatenate([acc_ref[0][:, :H], acc_ref[1][:, :H]], axis=1)
    gs = jnp.concatenate([acc_ref[0][:, H:], acc_ref[1][:, H:]], axis=1)
    agg = agg + aggl_ref[...]
    gs = gs + gl_ref[...] + 1e-6
    hn_ref[...] = h_ref[...] + jnp.maximum(hu_ref[...] + agg / gs, 0.0)


def _prednode_body(h_ref, wa_ref, wb_ref, hs_ref, hd_ref):
    h = h_ref[...]
    hs_ref[...] = jnp.dot(h, wa_ref[...], preferred_element_type=F32)
    hd_ref[...] = jnp.dot(h, wb_ref[...], preferred_element_type=F32)


def _full(shape):
    return pl.BlockSpec(shape, lambda i: tuple(0 for _ in shape))


def _edge_body(srcg_ref, dstg_ref, dstl_ref, hav_ref, hb2_ref, etc_ref, zer_ref,
               r_ref, acc_ref,
               sg2, dg2, dl0, dl1,
               av0, b0, e0, mg0,
               av1, b1, e1, mg1,
               acc_sh, sgs0, sgs1, sws0, sws1):
    c = lax.axis_index("c")
    s = lax.axis_index("s")
    r0 = s * RPT

    pltpu.sync_copy(zer_ref.at[pl.ds(r0, RPT)], acc_sh.at[pl.ds(r0, RPT)])
    plsc.subcore_barrier()

    bufs = ((av0, b0, e0, mg0, dl0, sgs0, sws0),
            (av1, b1, e1, mg1, dl1, sgs1, sws1))

    def gathers(ch, gi, p):
        av, bb, eb, _, dl, sg, _ = bufs[p]
        ebase = s * EPT + ch * CK
        return (pltpu.async_copy(hav_ref.at[sg2.at[gi]], av, sg),
                pltpu.async_copy(hb2_ref.at[dg2.at[gi]], bb, sg),
                pltpu.async_copy(etc_ref.at[c, pl.ds(ebase, CK)], eb, sg),
                pltpu.async_copy(dstl_ref.at[s * NCH + ch], dl, sg))

    def compute(p):
        av, bb, eb, mg, _, _, _ = bufs[p]

        def rowfn(row):
            for j in range(4):
                sl = pl.ds(j * 16, 16)
                slv = pl.ds(H + j * 16, 16)
                en = av[row, sl] + bb[row, sl] + eb[row, sl]
                eb[row, sl] = jnp.maximum(en, 0.0)
                gt = 1.0 / (1.0 + jnp.exp(-en))
                mg[row, sl] = gt * av[row, slv]
                mg[row, slv] = gt

        plsc.parallel_loop(0, CK, unroll=2)(rowfn)

    def writes(ch, gi, p):
        _, _, eb, mg, dl, _, sw = bufs[p]
        ebase = s * EPT + ch * CK
        w = pltpu.async_copy(eb, r_ref.at[c, pl.ds(ebase, CK)], sw)
        pltpu.sync_copy(mg, acc_sh.at[dl], add=True)
        return (w,)

    def group(g, cc):
        gbase = s * NCH + g * GE
        pltpu.sync_copy(srcg_ref.at[pl.ds(c * ECH + gbase, GE)], sg2)
        pltpu.sync_copy(dstg_ref.at[pl.ds(c * ECH + gbase, GE)], dg2)

        def body(i, cc2):
            gi0 = 2 * i
            gi1 = 2 * i + 1
            ch0 = g * GE + gi0
            ch1 = g * GE + gi1
            h0 = gathers(ch0, gi0, 0)
            h1 = gathers(ch1, gi1, 1)
            for hh in h0:
                hh.wait()
            compute(0)
            w0 = writes(ch0, gi0, 0)
            for hh in h1:
                hh.wait()
            compute(1)
            w1 = writes(ch1, gi1, 1)
            for ww in w0:
                ww.wait()
            for ww in w1:
                ww.wait()
            return cc2

        lax.fori_loop(0, GE // 2, body, 0)
        return cc

    lax.fori_loop(0, NCH // GE, group, 0)

    plsc.subcore_barrier()
    pltpu.sync_copy(acc_sh.at[pl.ds(r0, RPT)], acc_ref.at[c, pl.ds(r0, RPT)])


def _edge_phase(srcg, dstg, dstl, hav, hb2, etc, zer):
    fn = pl.kernel(
        _edge_body,
        mesh=_mesh,
        out_type=[
            jax.ShapeDtypeStruct((2, E, H), F32),
            jax.ShapeDtypeStruct((2, NP, 2 * H), F32),
        ],
        compiler_params=_SC_PARAMS,
        scratch_types=[
            pltpu.VMEM((GE, CK), jnp.int32),
            pltpu.VMEM((GE, CK), jnp.int32),
            pltpu.VMEM((CK,), jnp.int32),
            pltpu.VMEM((CK,), jnp.int32),
            pltpu.VMEM((CK, 2 * H), F32),
            pltpu.VMEM((CK, H), F32),
            pltpu.VMEM((CK, H), F32),
            pltpu.VMEM((CK, 2 * H), F32),
            pltpu.VMEM((CK, 2 * H), F32),
            pltpu.VMEM((CK, H), F32),
            pltpu.VMEM((CK, H), F32),
            pltpu.VMEM((CK, 2 * H), F32),
            pltpu.VMEM_SHARED((NP, 2 * H), F32),
            pltpu.SemaphoreType.DMA,
            pltpu.SemaphoreType.DMA,
            pltpu.SemaphoreType.DMA,
            pltpu.SemaphoreType.DMA,
        ],
    )
    return fn(srcg, dstg, dstl, hav, hb2, etc, zer)


def _pred_body(srcl_ref, dstl_ref, hs_ref, hd_ref, etp_ref, w_ref,
               out_ref,
               sl2, dl2, w_v,
               bs0, bd0, be0, ov0,
               bs1, bd1, be1, ov1,
               sgs0, sgs1, sws0, sws1):
    c = lax.axis_index("c")
    s = lax.axis_index("s")
    wid = s * NC + c
    base = wid * EPW
    pltpu.sync_copy(w_ref, w_v)

    bufs = ((bs0, bd0, be0, ov0, sgs0, sws0),
            (bs1, bd1, be1, ov1, sgs1, sws1))

    def gathers(ch, gi, p):
        bs, bd, be, _, sg, _ = bufs[p]
        return (pltpu.async_copy(hs_ref.at[sl2.at[gi]], bs.at[pl.ds(0, CK)], sg),
                pltpu.async_copy(hd_ref.at[dl2.at[gi]], bd.at[pl.ds(0, CK)], sg),
                pltpu.async_copy(etp_ref.at[pl.ds(base + ch * CK, CK)],
                                 be.at[pl.ds(0, CK)], sg))

    lanes = lax.iota(jnp.int32, 16)

    def compute(p):
        bs, bd, be, ov, _, _ = bufs[p]

        def gfn(g):
            vec = jnp.zeros((16,), F32)
            for r in range(16):
                row = g * 16 + r
                acc = jnp.zeros((16,), F32)
                for j in range(4):
                    sl = pl.ds(j * 16, 16)
                    z = jnp.maximum(bs[row, sl] + bd[row, sl] + be[row, sl], 0.0)
                    acc = acc + z * w_v[sl]
                vec = jnp.where(lanes == r, jnp.sum(acc), vec)
            ov[pl.ds(g * 16, 16)] = vec

        def gfn_i(g, cc):
            gfn(g)
            return cc

        lax.fori_loop(0, PCK // 16, gfn_i, 0)

    def group(g, cc):
        gbase = wid * NCHP + g * GP
        pltpu.sync_copy(srcl_ref.at[pl.ds(gbase, GP)], sl2)
        pltpu.sync_copy(dstl_ref.at[pl.ds(gbase, GP)], dl2)

        def body(i, cc2):
            gi0 = 2 * i
            gi1 = 2 * i + 1
            ch0 = g * GP + gi0
            ch1 = g * GP + gi1
            h0 = gathers(ch0, gi0, 0)
            h1 = gathers(ch1, gi1, 1)
            for hh in h0:
                hh.wait()
            compute(0)
            w0 = pltpu.async_copy(ov0.at[pl.ds(0, CK)],
                                  out_ref.at[pl.ds(base + ch0 * CK, CK)],
                                  bufs[0][5])
            for hh in h1:
                hh.wait()
            compute(1)
            w1 = pltpu.async_copy(ov1.at[pl.ds(0, CK)],
                                  out_ref.at[pl.ds(base + ch1 * CK, CK)],
                                  bufs[1][5])
            w0.wait()
            w1.wait()
            return cc2

        lax.fori_loop(0, GP // 2, body, 0)
        return cc

    lax.fori_loop(0, NCHP // GP, group, 0)


def _pred_phase(srcl, dstl, hs1, hd1, etp, w2):
    fn = pl.kernel(
        _pred_body,
        mesh=_mesh,
        out_type=jax.ShapeDtypeStruct((E,), F32),
        compiler_params=_SC_PARAMS,
        scratch_types=[
            pltpu.VMEM((GP, CK), jnp.int32),
            pltpu.VMEM((GP, CK), jnp.int32),
            pltpu.VMEM((H,), F32),
            pltpu.VMEM((PCK, H), F32),
            pltpu.VMEM((PCK, H), F32),
            pltpu.VMEM((PCK, H), F32),
            pltpu.VMEM((PCK,), F32),
            pltpu.VMEM((PCK, H), F32),
            pltpu.VMEM((PCK, H), F32),
            pltpu.VMEM((PCK, H), F32),
            pltpu.VMEM((PCK,), F32),
            pltpu.SemaphoreType.DMA,
            pltpu.SemaphoreType.DMA,
            pltpu.SemaphoreType.DMA,
            pltpu.SemaphoreType.DMA,
        ],
    )
    return fn(srcl, dstl, hs1, hd1, etp, w2)


def kernel(x, e, pe, edge_index, W_pe, b_pe, W1e, b1e, W2e, b2e,
           A, B, C, U, V, Wp1, bp1, Wp2, bp2):
    del x
    src = edge_index[0]
    dst = edge_index[1]
    srcg = jnp.concatenate([src, src + N]).reshape(2 * ECH, CK)
    dstg = jnp.concatenate([dst, dst + N]).reshape(2 * ECH, CK)
    dstl = dst.reshape(ECH, CK)
    srcl = src.reshape(ECH, CK)
    zer_np = jnp.zeros((NP, 2 * H), F32)
    zer_n = jnp.zeros((N, D), F32)

    bn = 2000
    nb_n = N // bn
    bk = 2000
    nb_e = E // bk

    h = pl.pallas_call(
        _enc_h_body,
        grid=(nb_n,),
        in_specs=[
            pl.BlockSpec((bn, 18), lambda i: (i, 0)),
            _full((18, D)),
            _full((1, D)),
        ],
        out_specs=pl.BlockSpec((bn, D), lambda i: (i, 0)),
        out_shape=jax.ShapeDtypeStruct((N, D), F32),
    )(pe, W_pe, b_pe.reshape(1, D))

    et2, etc = pl.pallas_call(
        _enc_et_body,
        grid=(nb_e,),
        in_specs=[
            pl.BlockSpec((bk, 16), lambda i: (i, 0)),
            _full((16, 64)),
            _full((1, 64)),
            _full((64, D)),
            _full((1, D)),
            _full((D, D)),
        ],
        out_specs=[
            pl.BlockSpec((bk, D), lambda i: (i, 0)),
            pl.BlockSpec((2, bk, H), lambda i: (0, i, 0)),
        ],
        out_shape=[
            jax.ShapeDtypeStruct((E, D), F32),
            jax.ShapeDtypeStruct((2, E, H), F32),
        ],
    )(e, W1e, b1e.reshape(1, 64), W2e, b2e.reshape(1, D), C[0])

    etl = zer_n
    for l in range(L):
        if l > 0:
            et2, etc = pl.pallas_call(
                _etc_body,
                grid=(nb_e,),
                in_specs=[
                    pl.BlockSpec((bk, D), lambda i: (i, 0)),
                    pl.BlockSpec((2, bk, H), lambda i: (0, i, 0)),
                    _full((D, D)),
                ],
                out_specs=[
                    pl.BlockSpec((bk, D), lambda i: (i, 0)),
                    pl.BlockSpec((2, bk, H), lambda i: (0, i, 0)),
                ],
                out_shape=[
                    jax.ShapeDtypeStruct((E, D), F32),
                    jax.ShapeDtypeStruct((2, E, H), F32),
                ],
            )(et2, r_split, C[l])

        hav, hb2, hu, gl, aggl, etl = pl.pallas_call(
            _tables_body,
            grid=(nb_n,),
            in_specs=[
                pl.BlockSpec((bn, D), lambda i: (i, 0)),
                pl.BlockSpec((bn, D), lambda i: (i, 0)),
                _full((D, D)), _full((D, D)), _full((D, D)),
                _full((D, D)), _full((D, D)),
            ],
            out_specs=[
                pl.BlockSpec((2, bn, D), lambda i: (0, i, 0)),
                pl.BlockSpec((2, bn, H), lambda i: (0, i, 0)),
                pl.BlockSpec((bn, D), lambda i: (i, 0)),
                pl.BlockSpec((bn, D), lambda i: (i, 0)),
                pl.BlockSpec((bn, D), lambda i: (i, 0)),
                pl.BlockSpec((bn, D), lambda i: (i, 0)),
            ],
            out_shape=[
                jax.ShapeDtypeStruct((2, N, D), F32),
                jax.ShapeDtypeStruct((2, N, H), F32),
                jax.ShapeDtypeStruct((N, D), F32),
                jax.ShapeDtypeStruct((N, D), F32),
                jax.ShapeDtypeStruct((N, D), F32),
                jax.ShapeDtypeStruct((N, D), F32),
            ],
        )(h, etl, A[l], B[l], V[l], U[l], C[l])

        r_split, acc2 = _edge_phase(
            srcg, dstg, dstl,
            hav.reshape(2 * N, D),
            hb2.reshape(2 * N, H),
            etc,
            zer_np,
        )

        h = pl.pallas_call(
            _hupd_body,
            grid=(nb_n,),
            in_specs=[
                pl.BlockSpec((bn, D), lambda i: (i, 0)),
                pl.BlockSpec((bn, D), lambda i: (i, 0)),
                pl.BlockSpec((bn, D), lambda i: (i, 0)),
                pl.BlockSpec((bn, D), lambda i: (i, 0)),
                pl.BlockSpec((2, bn, 2 * H), lambda i: (0, i, 0)),
            ],
            out_specs=pl.BlockSpec((bn, D), lambda i: (i, 0)),
            out_shape=jax.ShapeDtypeStruct((N, D), F32),
        )(h, hu, aggl, gl, acc2)

    hs1, hd1 = pl.pallas_call(
        _prednode_body,
        grid=(nb_n,),
        in_specs=[
            pl.BlockSpec((bn, D), lambda i: (i, 0)),
            _full((D, H)),
            _full((D, H)),
        ],
        out_specs=[
            pl.BlockSpec((bn, H), lambda i: (i, 0)),
            pl.BlockSpec((bn, H), lambda i: (i, 0)),
        ],
        out_shape=[
            jax.ShapeDtypeStruct((N, H), F32),
            jax.ShapeDtypeStruct((N, H), F32),
        ],
    )(h, Wp1[:D], Wp1[D:2 * D])

    etp = pl.pallas_call(
        _etp_body,
        grid=(nb_e,),
        in_specs=[
            pl.BlockSpec((bk, D), lambda i: (i, 0)),
            pl.BlockSpec((2, bk, H), lambda i: (0, i, 0)),
            _full((D, H)),
            _full((1, H)),
        ],
        out_specs=pl.BlockSpec((bk, H), lambda i: (i, 0)),
        out_shape=jax.ShapeDtypeStruct((E, H), F32),
    )(et2, r_split, Wp1[2 * D:], bp1.reshape(1, H))

    scores = _pred_phase(srcl, dstl, hs1, hd1, etp, Wp2.reshape(H))
    return scores.reshape(E, 1) + bp2


# speedup vs baseline: 2.9197x; 1.0241x over previous
import jax
import jax.numpy as jnp
from jax import lax
from jax.experimental import pallas as pl
from jax.experimental.pallas import tpu as pltpu
from jax.experimental.pallas import tpu_sc as plsc

F32 = jnp.float32

N = 10000
E = 320000
D = 128
H = 64
NP = 10240
L = 3

NS = 16
NC = 2
CK = 40
EPT = E // NS
EPW = E // (NS * NC)
RPT = NP // NS
NCH = EPT // CK
ECH = E // CK
NCHP = EPW // CK
GE = 20
GP = 50
PCK = 48

_mesh = plsc.VectorSubcoreMesh(core_axis_name="c", subcore_axis_name="s")

_SC_PARAMS = pltpu.CompilerParams(
    use_tc_tiling_on_sc=False, needs_layout_passes=False)


def _sigmoid(x):
    return 1.0 / (1.0 + jnp.exp(-x))


def _enc_h_body(pe_ref, wpe_ref, bpe_ref, h_ref):
    h_ref[...] = (
        jnp.dot(pe_ref[...], wpe_ref[...], preferred_element_type=F32)
        + bpe_ref[...]
    )


def _enc_et_body(e_ref, w1_ref, b1_ref, w2_ref, b2_ref, c0_ref, et_ref, etc_ref):
    t = jnp.maximum(
        jnp.dot(e_ref[...], w1_ref[...], preferred_element_type=F32) + b1_ref[...],
        0.0,
    )
    et = jnp.dot(t, w2_ref[...], preferred_element_type=F32) + b2_ref[...]
    et_ref[...] = et
    etc = jnp.dot(et, c0_ref[...], preferred_element_type=F32)
    etc_ref[0] = etc[:, :H]
    etc_ref[1] = etc[:, H:]


def _tables_body(h_ref, etl_ref, a_ref, b_ref, v_ref, u_ref, c_ref,
                 hav_ref, hb2_ref, hu_ref, gl_ref, aggl_ref, etln_ref):
    h = h_ref[...]
    hA = jnp.dot(h, a_ref[...], preferred_element_type=F32)
    hB = jnp.dot(h, b_ref[...], preferred_element_type=F32)
    hV = jnp.dot(h, v_ref[...], preferred_element_type=F32)
    hU = jnp.dot(h, u_ref[...], preferred_element_type=F32)
    eL = hA + hB + jnp.dot(etl_ref[...], c_ref[...], preferred_element_type=F32)
    gL = _sigmoid(eL)
    hav_ref[0] = jnp.concatenate([hA[:, :H], hV[:, :H]], axis=1)
    hav_ref[1] = jnp.concatenate([hA[:, H:], hV[:, H:]], axis=1)
    hb2_ref[0] = hB[:, :H]
    hb2_ref[1] = hB[:, H:]
    hu_ref[...] = hU
    gl_ref[...] = gL
    aggl_ref[...] = gL * hV
    etln_ref[...] = etl_ref[...] + jnp.maximum(eL, 0.0)


def _etc_body(et2_ref, r_ref, c_ref, et2n_ref, etc_ref):
    rfull = jnp.concatenate([r_ref[0], r_ref[1]], axis=1)
    etn = et2_ref[...] + rfull
    et2n_ref[...] = etn
    etc = jnp.dot(etn, c_ref[...], preferred_element_type=F32)
    etc_ref[0] = etc[:, :H]
    etc_ref[1] = etc[:, H:]


def _etp_body(et2_ref, r_ref, w_ref, b_ref, etp_ref):
    etn = et2_ref[...] + jnp.concatenate([r_ref[0], r_ref[1]], axis=1)
    etp_ref[...] = jnp.dot(etn, w_ref[...], preferred_element_type=F32) + b_ref[...]


def _hupd_body(h_ref, hu_ref, aggl_ref, gl_ref, acc_ref, hn_ref):
    agg = jnp.concatenate([acc_ref[0][:, :H], acc_ref[1][:, :H]], axis=1)
    gs = jnp.concatenate([acc_ref[0][:, H:], acc_ref[1][:, H:]], axis=1)
    agg = agg + aggl_ref[...]
    gs = gs + gl_ref[...] + 1e-6
    hn_ref[...] = h_ref[...] + jnp.maximum(hu_ref[...] + agg / gs, 0.0)


def _prednode_body(h_ref, wa_ref, wb_ref, hs_ref, hd_ref):
    h = h_ref[...]
    hs_ref[...] = jnp.dot(h, wa_ref[...], preferred_element_type=F32)
    hd_ref[...] = jnp.dot(h, wb_ref[...], preferred_element_type=F32)


def _full(shape):
    return pl.BlockSpec(shape, lambda i: tuple(0 for _ in shape))


def _edge_body(srcg_ref, dstg_ref, dstl_ref, hav_ref, hb2_ref, etc_ref, zer_ref,
               r_ref, acc_ref,
               sg2, dg2, dl0, dl1,
               av0, b0, e0, mg0,
               av1, b1, e1, mg1,
               acc_sh, sgs0, sgs1, sws0, sws1):
    c = lax.axis_index("c")
    s = lax.axis_index("s")
    r0 = s * RPT

    pltpu.sync_copy(zer_ref.at[pl.ds(r0, RPT)], acc_sh.at[pl.ds(r0, RPT)])
    plsc.subcore_barrier()

    bufs = ((av0, b0, e0, mg0, dl0, sgs0, sws0),
            (av1, b1, e1, mg1, dl1, sgs1, sws1))

    def gathers(ch, gi, p):
        av, bb, eb, _, dl, sg, _ = bufs[p]
        ebase = s * EPT + ch * CK
        return (pltpu.async_copy(hav_ref.at[sg2.at[gi]], av, sg),
                pltpu.async_copy(hb2_ref.at[dg2.at[gi]], bb, sg),
                pltpu.async_copy(etc_ref.at[c, pl.ds(ebase, CK)], eb, sg),
                pltpu.async_copy(dstl_ref.at[s * NCH + ch], dl, sg))

    def compute(p):
        av, bb, eb, mg, _, _, _ = bufs[p]

        def rowfn(row):
            for j in range(4):
                sl = pl.ds(j * 16, 16)
                slv = pl.ds(H + j * 16, 16)
                en = av[row, sl] + bb[row, sl] + eb[row, sl]
                eb[row, sl] = jnp.maximum(en, 0.0)
                gt = 1.0 / (1.0 + jnp.exp(-en))
                mg[row, sl] = gt * av[row, slv]
                mg[row, slv] = gt

        plsc.parallel_loop(0, CK, unroll=4)(rowfn)

    def writes(ch, gi, p):
        _, _, eb, mg, dl, _, sw = bufs[p]
        ebase = s * EPT + ch * CK
        w = pltpu.async_copy(eb, r_ref.at[c, pl.ds(ebase, CK)], sw)
        pltpu.sync_copy(mg, acc_sh.at[dl], add=True)
        return (w,)

    def group(g, cc):
        gbase = s * NCH + g * GE
        pltpu.sync_copy(srcg_ref.at[pl.ds(c * ECH + gbase, GE)], sg2)
        pltpu.sync_copy(dstg_ref.at[pl.ds(c * ECH + gbase, GE)], dg2)

        def body(i, cc2):
            gi0 = 2 * i
            gi1 = 2 * i + 1
            ch0 = g * GE + gi0
            ch1 = g * GE + gi1
            h0 = gathers(ch0, gi0, 0)
            h1 = gathers(ch1, gi1, 1)
            for hh in h0:
                hh.wait()
            compute(0)
            w0 = writes(ch0, gi0, 0)
            for hh in h1:
                hh.wait()
            compute(1)
            w1 = writes(ch1, gi1, 1)
            for ww in w0:
                ww.wait()
            for ww in w1:
                ww.wait()
            return cc2

        lax.fori_loop(0, GE // 2, body, 0)
        return cc

    lax.fori_loop(0, NCH // GE, group, 0)

    plsc.subcore_barrier()
    pltpu.sync_copy(acc_sh.at[pl.ds(r0, RPT)], acc_ref.at[c, pl.ds(r0, RPT)])


def _edge_phase(srcg, dstg, dstl, hav, hb2, etc, zer):
    fn = pl.kernel(
        _edge_body,
        mesh=_mesh,
        out_type=[
            jax.ShapeDtypeStruct((2, E, H), F32),
            jax.ShapeDtypeStruct((2, NP, 2 * H), F32),
        ],
        compiler_params=_SC_PARAMS,
        scratch_types=[
            pltpu.VMEM((GE, CK), jnp.int32),
            pltpu.VMEM((GE, CK), jnp.int32),
            pltpu.VMEM((CK,), jnp.int32),
            pltpu.VMEM((CK,), jnp.int32),
            pltpu.VMEM((CK, 2 * H), F32),
            pltpu.VMEM((CK, H), F32),
            pltpu.VMEM((CK, H), F32),
            pltpu.VMEM((CK, 2 * H), F32),
            pltpu.VMEM((CK, 2 * H), F32),
            pltpu.VMEM((CK, H), F32),
            pltpu.VMEM((CK, H), F32),
            pltpu.VMEM((CK, 2 * H), F32),
            pltpu.VMEM_SHARED((NP, 2 * H), F32),
            pltpu.SemaphoreType.DMA,
            pltpu.SemaphoreType.DMA,
            pltpu.SemaphoreType.DMA,
            pltpu.SemaphoreType.DMA,
        ],
    )
    return fn(srcg, dstg, dstl, hav, hb2, etc, zer)


def _pred_body(srcl_ref, dstl_ref, hs_ref, hd_ref, etp_ref, w_ref,
               out_ref,
               sl2, dl2, w_v,
               bs0, bd0, be0, ov0,
               bs1, bd1, be1, ov1,
               sgs0, sgs1, sws0, sws1):
    c = lax.axis_index("c")
    s = lax.axis_index("s")
    wid = s * NC + c
    base = wid * EPW
    pltpu.sync_copy(w_ref, w_v)

    bufs = ((bs0, bd0, be0, ov0, sgs0, sws0),
            (bs1, bd1, be1, ov1, sgs1, sws1))

    def gathers(ch, gi, p):
        bs, bd, be, _, sg, _ = bufs[p]
        return (pltpu.async_copy(hs_ref.at[sl2.at[gi]], bs.at[pl.ds(0, CK)], sg),
                pltpu.async_copy(hd_ref.at[dl2.at[gi]], bd.at[pl.ds(0, CK)], sg),
                pltpu.async_copy(etp_ref.at[pl.ds(base + ch * CK, CK)],
                                 be.at[pl.ds(0, CK)], sg))

    lanes = lax.iota(jnp.int32, 16)

    def compute(p):
        bs, bd, be, ov, _, _ = bufs[p]

        def gfn(g):
            vec = jnp.zeros((16,), F32)
            for r in range(16):
                row = g * 16 + r
                acc = jnp.zeros((16,), F32)
                for j in range(4):
                    sl = pl.ds(j * 16, 16)
                    z = jnp.maximum(bs[row, sl] + bd[row, sl] + be[row, sl], 0.0)
                    acc = acc + z * w_v[sl]
                vec = jnp.where(lanes == r, jnp.sum(acc), vec)
            ov[pl.ds(g * 16, 16)] = vec

        def gfn_i(g, cc):
            gfn(g)
            return cc

        lax.fori_loop(0, PCK // 16, gfn_i, 0)

    def group(g, cc):
        gbase = wid * NCHP + g * GP
        pltpu.sync_copy(srcl_ref.at[pl.ds(gbase, GP)], sl2)
        pltpu.sync_copy(dstl_ref.at[pl.ds(gbase, GP)], dl2)

        def body(i, cc2):
            gi0 = 2 * i
            gi1 = 2 * i + 1
            ch0 = g * GP + gi0
            ch1 = g * GP + gi1
            h0 = gathers(ch0, gi0, 0)
            h1 = gathers(ch1, gi1, 1)
            for hh in h0:
                hh.wait()
            compute(0)
            w0 = pltpu.async_copy(ov0.at[pl.ds(0, CK)],
                                  out_ref.at[pl.ds(base + ch0 * CK, CK)],
                                  bufs[0][5])
            for hh in h1:
                hh.wait()
            compute(1)
            w1 = pltpu.async_copy(ov1.at[pl.ds(0, CK)],
                                  out_ref.at[pl.ds(base + ch1 * CK, CK)],
                                  bufs[1][5])
            w0.wait()
            w1.wait()
            return cc2

        lax.fori_loop(0, GP // 2, body, 0)
        return cc

    lax.fori_loop(0, NCHP // GP, group, 0)


def _pred_phase(srcl, dstl, hs1, hd1, etp, w2):
    fn = pl.kernel(
        _pred_body,
        mesh=_mesh,
        out_type=jax.ShapeDtypeStruct((E,), F32),
        compiler_params=_SC_PARAMS,
        scratch_types=[
            pltpu.VMEM((GP, CK), jnp.int32),
            pltpu.VMEM((GP, CK), jnp.int32),
            pltpu.VMEM((H,), F32),
            pltpu.VMEM((PCK, H), F32),
            pltpu.VMEM((PCK, H), F32),
            pltpu.VMEM((PCK, H), F32),
            pltpu.VMEM((PCK,), F32),
            pltpu.VMEM((PCK, H), F32),
            pltpu.VMEM((PCK, H), F32),
            pltpu.VMEM((PCK, H), F32),
            pltpu.VMEM((PCK,), F32),
            pltpu.SemaphoreType.DMA,
            pltpu.SemaphoreType.DMA,
            pltpu.SemaphoreType.DMA,
            pltpu.SemaphoreType.DMA,
        ],
    )
    return fn(srcl, dstl, hs1, hd1, etp, w2)


def kernel(x, e, pe, edge_index, W_pe, b_pe, W1e, b1e, W2e, b2e,
           A, B, C, U, V, Wp1, bp1, Wp2, bp2):
    del x
    src = edge_index[0]
    dst = edge_index[1]
    srcg = jnp.concatenate([src, src + N]).reshape(2 * ECH, CK)
    dstg = jnp.concatenate([dst, dst + N]).reshape(2 * ECH, CK)
    dstl = dst.reshape(ECH, CK)
    srcl = src.reshape(ECH, CK)
    zer_np = jnp.zeros((NP, 2 * H), F32)
    zer_n = jnp.zeros((N, D), F32)

    bn = 2000
    nb_n = N // bn
    bk = 4000
    nb_e = E // bk

    h = pl.pallas_call(
        _enc_h_body,
        grid=(nb_n,),
        in_specs=[
            pl.BlockSpec((bn, 18), lambda i: (i, 0)),
            _full((18, D)),
            _full((1, D)),
        ],
        out_specs=pl.BlockSpec((bn, D), lambda i: (i, 0)),
        out_shape=jax.ShapeDtypeStruct((N, D), F32),
    )(pe, W_pe, b_pe.reshape(1, D))

    et2, etc = pl.pallas_call(
        _enc_et_body,
        grid=(nb_e,),
        in_specs=[
            pl.BlockSpec((bk, 16), lambda i: (i, 0)),
            _full((16, 64)),
            _full((1, 64)),
            _full((64, D)),
            _full((1, D)),
            _full((D, D)),
        ],
        out_specs=[
            pl.BlockSpec((bk, D), lambda i: (i, 0)),
            pl.BlockSpec((2, bk, H), lambda i: (0, i, 0)),
        ],
        out_shape=[
            jax.ShapeDtypeStruct((E, D), F32),
            jax.ShapeDtypeStruct((2, E, H), F32),
        ],
    )(e, W1e, b1e.reshape(1, 64), W2e, b2e.reshape(1, D), C[0])

    etl = zer_n
    for l in range(L):
        if l > 0:
            et2, etc = pl.pallas_call(
                _etc_body,
                grid=(nb_e,),
                in_specs=[
                    pl.BlockSpec((bk, D), lambda i: (i, 0)),
                    pl.BlockSpec((2, bk, H), lambda i: (0, i, 0)),
                    _full((D, D)),
                ],
                out_specs=[
                    pl.BlockSpec((bk, D), lambda i: (i, 0)),
                    pl.BlockSpec((2, bk, H), lambda i: (0, i, 0)),
                ],
                out_shape=[
                    jax.ShapeDtypeStruct((E, D), F32),
                    jax.ShapeDtypeStruct((2, E, H), F32),
                ],
            )(et2, r_split, C[l])

        hav, hb2, hu, gl, aggl, etl = pl.pallas_call(
            _tables_body,
            grid=(nb_n,),
            in_specs=[
                pl.BlockSpec((bn, D), lambda i: (i, 0)),
                pl.BlockSpec((bn, D), lambda i: (i, 0)),
                _full((D, D)), _full((D, D)), _full((D, D)),
                _full((D, D)), _full((D, D)),
            ],
            out_specs=[
                pl.BlockSpec((2, bn, D), lambda i: (0, i, 0)),
                pl.BlockSpec((2, bn, H), lambda i: (0, i, 0)),
                pl.BlockSpec((bn, D), lambda i: (i, 0)),
                pl.BlockSpec((bn, D), lambda i: (i, 0)),
                pl.BlockSpec((bn, D), lambda i: (i, 0)),
                pl.BlockSpec((bn, D), lambda i: (i, 0)),
            ],
            out_shape=[
                jax.ShapeDtypeStruct((2, N, D), F32),
                jax.ShapeDtypeStruct((2, N, H), F32),
                jax.ShapeDtypeStruct((N, D), F32),
                jax.ShapeDtypeStruct((N, D), F32),
                jax.ShapeDtypeStruct((N, D), F32),
                jax.ShapeDtypeStruct((N, D), F32),
            ],
        )(h, etl, A[l], B[l], V[l], U[l], C[l])

        r_split, acc2 = _edge_phase(
            srcg, dstg, dstl,
            hav.reshape(2 * N, D),
            hb2.reshape(2 * N, H),
            etc,
            zer_np,
        )

        h = pl.pallas_call(
            _hupd_body,
            grid=(nb_n,),
            in_specs=[
                pl.BlockSpec((bn, D), lambda i: (i, 0)),
                pl.BlockSpec((bn, D), lambda i: (i, 0)),
                pl.BlockSpec((bn, D), lambda i: (i, 0)),
                pl.BlockSpec((bn, D), lambda i: (i, 0)),
                pl.BlockSpec((2, bn, 2 * H), lambda i: (0, i, 0)),
            ],
            out_specs=pl.BlockSpec((bn, D), lambda i: (i, 0)),
            out_shape=jax.ShapeDtypeStruct((N, D), F32),
        )(h, hu, aggl, gl, acc2)

    hs1, hd1 = pl.pallas_call(
        _prednode_body,
        grid=(nb_n,),
        in_specs=[
            pl.BlockSpec((bn, D), lambda i: (i, 0)),
            _full((D, H)),
            _full((D, H)),
        ],
        out_specs=[
            pl.BlockSpec((bn, H), lambda i: (i, 0)),
            pl.BlockSpec((bn, H), lambda i: (i, 0)),
        ],
        out_shape=[
            jax.ShapeDtypeStruct((N, H), F32),
            jax.ShapeDtypeStruct((N, H), F32),
        ],
    )(h, Wp1[:D], Wp1[D:2 * D])

    etp = pl.pallas_call(
        _etp_body,
        grid=(nb_e,),
        in_specs=[
            pl.BlockSpec((bk, D), lambda i: (i, 0)),
            pl.BlockSpec((2, bk, H), lambda i: (0, i, 0)),
            _full((D, H)),
            _full((1, H)),
        ],
        out_specs=pl.BlockSpec((bk, H), lambda i: (i, 0)),
        out_shape=jax.ShapeDtypeStruct((E, H), F32),
    )(et2, r_split, Wp1[2 * D:], bp1.reshape(1, H))

    scores = _pred_phase(srcl, dstl, hs1, hd1, etp, Wp2.reshape(H))
    return scores.reshape(E, 1) + bp2


# speedup vs baseline: 2.9553x; 1.0122x over previous
import jax
import jax.numpy as jnp
from jax import lax
from jax.experimental import pallas as pl
from jax.experimental.pallas import tpu as pltpu
from jax.experimental.pallas import tpu_sc as plsc

F32 = jnp.float32

N = 10000
E = 320000
D = 128
H = 64
NP = 10240
L = 3

NS = 16
NC = 2
CK = 40
EPT = E // NS
EPW = E // (NS * NC)
RPT = NP // NS
NCH = EPT // CK
ECH = E // CK
NCHP = EPW // CK
GE = 100
GP = 50
PCK = 48

_mesh = plsc.VectorSubcoreMesh(core_axis_name="c", subcore_axis_name="s")

_SC_PARAMS = pltpu.CompilerParams(
    use_tc_tiling_on_sc=False, needs_layout_passes=False)


def _sigmoid(x):
    return 1.0 / (1.0 + jnp.exp(-x))


def _enc_h_body(pe_ref, wpe_ref, bpe_ref, h_ref):
    h_ref[...] = (
        jnp.dot(pe_ref[...], wpe_ref[...], preferred_element_type=F32)
        + bpe_ref[...]
    )


def _enc_et_body(e_ref, w1_ref, b1_ref, w2_ref, b2_ref, c0_ref, et_ref, etc_ref):
    t = jnp.maximum(
        jnp.dot(e_ref[...], w1_ref[...], preferred_element_type=F32) + b1_ref[...],
        0.0,
    )
    et = jnp.dot(t, w2_ref[...], preferred_element_type=F32) + b2_ref[...]
    et_ref[...] = et
    etc = jnp.dot(et, c0_ref[...], preferred_element_type=F32)
    etc_ref[0] = etc[:, :H]
    etc_ref[1] = etc[:, H:]


def _tables_body(h_ref, etl_ref, a_ref, b_ref, v_ref, u_ref, c_ref,
                 hav_ref, hb2_ref, hu_ref, gl_ref, aggl_ref, etln_ref):
    h = h_ref[...]
    hA = jnp.dot(h, a_ref[...], preferred_element_type=F32)
    hB = jnp.dot(h, b_ref[...], preferred_element_type=F32)
    hV = jnp.dot(h, v_ref[...], preferred_element_type=F32)
    hU = jnp.dot(h, u_ref[...], preferred_element_type=F32)
    eL = hA + hB + jnp.dot(etl_ref[...], c_ref[...], preferred_element_type=F32)
    gL = _sigmoid(eL)
    hav_ref[0] = jnp.concatenate([hA[:, :H], hV[:, :H]], axis=1)
    hav_ref[1] = jnp.concatenate([hA[:, H:], hV[:, H:]], axis=1)
    hb2_ref[0] = hB[:, :H]
    hb2_ref[1] = hB[:, H:]
    hu_ref[...] = hU
    gl_ref[...] = gL
    aggl_ref[...] = gL * hV
    etln_ref[...] = etl_ref[...] + jnp.maximum(eL, 0.0)


def _etc_body(et2_ref, r_ref, c_ref, et2n_ref, etc_ref):
    rfull = jnp.concatenate([r_ref[0], r_ref[1]], axis=1)
    etn = et2_ref[...] + rfull
    et2n_ref[...] = etn
    etc = jnp.dot(etn, c_ref[...], preferred_element_type=F32)
    etc_ref[0] = etc[:, :H]
    etc_ref[1] = etc[:, H:]


def _etp_body(et2_ref, r_ref, w_ref, b_ref, etp_ref):
    etn = et2_ref[...] + jnp.concatenate([r_ref[0], r_ref[1]], axis=1)
    etp_ref[...] = jnp.dot(etn, w_ref[...], preferred_element_type=F32) + b_ref[...]


def _hupd_body(h_ref, hu_ref, aggl_ref, gl_ref, acc_ref, hn_ref):
    agg = jnp.concatenate([acc_ref[0][:, :H], acc_ref[1][:, :H]], axis=1)
    gs = jnp.concatenate([acc_ref[0][:, H:], acc_ref[1][:, H:]], axis=1)
    agg = agg + aggl_ref[...]
    gs = gs + gl_ref[...] + 1e-6
    hn_ref[...] = h_ref[...] + jnp.maximum(hu_ref[...] + agg / gs, 0.0)


def _prednode_body(h_ref, wa_ref, wb_ref, hs_ref, hd_ref):
    h = h_ref[...]
    hs_ref[...] = jnp.dot(h, wa_ref[...], preferred_element_type=F32)
    hd_ref[...] = jnp.dot(h, wb_ref[...], preferred_element_type=F32)


def _full(shape):
    return pl.BlockSpec(shape, lambda i: tuple(0 for _ in shape))


def _edge_body(srcg_ref, dstg_ref, dstl_ref, hav_ref, hb2_ref, etc_ref, zer_ref,
               r_ref, acc_ref,
               sg2, dg2, dl0, dl1,
               av0, b0, e0, mg0,
               av1, b1, e1, mg1,
               acc_sh, sgs0, sgs1, sws0, sws1):
    c = lax.axis_index("c")
    s = lax.axis_index("s")
    r0 = s * RPT

    pltpu.sync_copy(zer_ref.at[pl.ds(r0, RPT)], acc_sh.at[pl.ds(r0, RPT)])
    plsc.subcore_barrier()

    bufs = ((av0, b0, e0, mg0, dl0, sgs0, sws0),
            (av1, b1, e1, mg1, dl1, sgs1, sws1))

    def gathers(ch, gi, p):
        av, bb, eb, _, dl, sg, _ = bufs[p]
        ebase = s * EPT + ch * CK
        return (pltpu.async_copy(hav_ref.at[sg2.at[gi]], av, sg),
                pltpu.async_copy(hb2_ref.at[dg2.at[gi]], bb, sg),
                pltpu.async_copy(etc_ref.at[c, pl.ds(ebase, CK)], eb, sg),
                pltpu.async_copy(dstl_ref.at[s * NCH + ch], dl, sg))

    def compute(p):
        av, bb, eb, mg, _, _, _ = bufs[p]

        def rowfn(row):
            for j in range(4):
                sl = pl.ds(j * 16, 16)
                slv = pl.ds(H + j * 16, 16)
                en = av[row, sl] + bb[row, sl] + eb[row, sl]
                eb[row, sl] = jnp.maximum(en, 0.0)
                gt = 1.0 / (1.0 + jnp.exp(-en))
                mg[row, sl] = gt * av[row, slv]
                mg[row, slv] = gt

        plsc.parallel_loop(0, CK, unroll=4)(rowfn)

    def writes(ch, gi, p):
        _, _, eb, mg, dl, _, sw = bufs[p]
        ebase = s * EPT + ch * CK
        w = pltpu.async_copy(eb, r_ref.at[c, pl.ds(ebase, CK)], sw)
        pltpu.sync_copy(mg, acc_sh.at[dl], add=True)
        return (w,)

    def group(g, cc):
        gbase = s * NCH + g * GE
        pltpu.sync_copy(srcg_ref.at[pl.ds(c * ECH + gbase, GE)], sg2)
        pltpu.sync_copy(dstg_ref.at[pl.ds(c * ECH + gbase, GE)], dg2)

        def body(i, cc2):
            gi0 = 2 * i
            gi1 = 2 * i + 1
            ch0 = g * GE + gi0
            ch1 = g * GE + gi1
            h0 = gathers(ch0, gi0, 0)
            h1 = gathers(ch1, gi1, 1)
            for hh in h0:
                hh.wait()
            compute(0)
            w0 = writes(ch0, gi0, 0)
            for hh in h1:
                hh.wait()
            compute(1)
            w1 = writes(ch1, gi1, 1)
            for ww in w0:
                ww.wait()
            for ww in w1:
                ww.wait()
            return cc2

        lax.fori_loop(0, GE // 2, body, 0)
        return cc

    lax.fori_loop(0, NCH // GE, group, 0)

    plsc.subcore_barrier()
    pltpu.sync_copy(acc_sh.at[pl.ds(r0, RPT)], acc_ref.at[c, pl.ds(r0, RPT)])


def _edge_phase(srcg, dstg, dstl, hav, hb2, etc, zer):
    fn = pl.kernel(
        _edge_body,
        mesh=_mesh,
        out_type=[
            jax.ShapeDtypeStruct((2, E, H), F32),
            jax.ShapeDtypeStruct((2, NP, 2 * H), F32),
        ],
        compiler_params=_SC_PARAMS,
        scratch_types=[
            pltpu.VMEM((GE, CK), jnp.int32),
            pltpu.VMEM((GE, CK), jnp.int32),
            pltpu.VMEM((CK,), jnp.int32),
            pltpu.VMEM((CK,), jnp.int32),
            pltpu.VMEM((CK, 2 * H), F32),
            pltpu.VMEM((CK, H), F32),
            pltpu.VMEM((CK, H), F32),
            pltpu.VMEM((CK, 2 * H), F32),
            pltpu.VMEM((CK, 2 * H), F32),
            pltpu.VMEM((CK, H), F32),
            pltpu.VMEM((CK, H), F32),
            pltpu.VMEM((CK, 2 * H), F32),
            pltpu.VMEM_SHARED((NP, 2 * H), F32),
            pltpu.SemaphoreType.DMA,
            pltpu.SemaphoreType.DMA,
            pltpu.SemaphoreType.DMA,
            pltpu.SemaphoreType.DMA,
        ],
    )
    return fn(srcg, dstg, dstl, hav, hb2, etc, zer)


def _pred_body(srcl_ref, dstl_ref, hs_ref, hd_ref, etp_ref, w_ref,
               out_ref,
               sl2, dl2, w_v,
               bs0, bd0, be0, ov0,
               bs1, bd1, be1, ov1,
               sgs0, sgs1, sws0, sws1):
    c = lax.axis_index("c")
    s = lax.axis_index("s")
    wid = s * NC + c
    base = wid * EPW
    pltpu.sync_copy(w_ref, w_v)

    bufs = ((bs0, bd0, be0, ov0, sgs0, sws0),
            (bs1, bd1, be1, ov1, sgs1, sws1))

    def gathers(ch, gi, p):
        bs, bd, be, _, sg, _ = bufs[p]
        return (pltpu.async_copy(hs_ref.at[sl2.at[gi]], bs.at[pl.ds(0, CK)], sg),
                pltpu.async_copy(hd_ref.at[dl2.at[gi]], bd.at[pl.ds(0, CK)], sg),
                pltpu.async_copy(etp_ref.at[pl.ds(base + ch * CK, CK)],
                                 be.at[pl.ds(0, CK)], sg))

    lanes = lax.iota(jnp.int32, 16)

    def compute(p):
        bs, bd, be, ov, _, _ = bufs[p]

        def gfn(g):
            vec = jnp.zeros((16,), F32)
            for r in range(16):
                row = g * 16 + r
                acc = jnp.zeros((16,), F32)
                for j in range(4):
                    sl = pl.ds(j * 16, 16)
                    z = jnp.maximum(bs[row, sl] + bd[row, sl] + be[row, sl], 0.0)
                    acc = acc + z * w_v[sl]
                vec = jnp.where(lanes == r, jnp.sum(acc), vec)
            ov[pl.ds(g * 16, 16)] = vec

        def gfn_i(g, cc):
            gfn(g)
            return cc

        lax.fori_loop(0, PCK // 16, gfn_i, 0)

    def group(g, cc):
        gbase = wid * NCHP + g * GP
        pltpu.sync_copy(srcl_ref.at[pl.ds(gbase, GP)], sl2)
        pltpu.sync_copy(dstl_ref.at[pl.ds(gbase, GP)], dl2)

        def body(i, cc2):
            gi0 = 2 * i
            gi1 = 2 * i + 1
            ch0 = g * GP + gi0
            ch1 = g * GP + gi1
            h0 = gathers(ch0, gi0, 0)
            h1 = gathers(ch1, gi1, 1)
            for hh in h0:
                hh.wait()
            compute(0)
            w0 = pltpu.async_copy(ov0.at[pl.ds(0, CK)],
                                  out_ref.at[pl.ds(base + ch0 * CK, CK)],
                                  bufs[0][5])
            for hh in h1:
                hh.wait()
            compute(1)
            w1 = pltpu.async_copy(ov1.at[pl.ds(0, CK)],
                                  out_ref.at[pl.ds(base + ch1 * CK, CK)],
                                  bufs[1][5])
            w0.wait()
            w1.wait()
            return cc2

        lax.fori_loop(0, GP // 2, body, 0)
        return cc

    lax.fori_loop(0, NCHP // GP, group, 0)


def _pred_phase(srcl, dstl, hs1, hd1, etp, w2):
    fn = pl.kernel(
        _pred_body,
        mesh=_mesh,
        out_type=jax.ShapeDtypeStruct((E,), F32),
        compiler_params=_SC_PARAMS,
        scratch_types=[
            pltpu.VMEM((GP, CK), jnp.int32),
            pltpu.VMEM((GP, CK), jnp.int32),
            pltpu.VMEM((H,), F32),
            pltpu.VMEM((PCK, H), F32),
            pltpu.VMEM((PCK, H), F32),
            pltpu.VMEM((PCK, H), F32),
            pltpu.VMEM((PCK,), F32),
            pltpu.VMEM((PCK, H), F32),
            pltpu.VMEM((PCK, H), F32),
            pltpu.VMEM((PCK, H), F32),
            pltpu.VMEM((PCK,), F32),
            pltpu.SemaphoreType.DMA,
            pltpu.SemaphoreType.DMA,
            pltpu.SemaphoreType.DMA,
            pltpu.SemaphoreType.DMA,
        ],
    )
    return fn(srcl, dstl, hs1, hd1, etp, w2)


def kernel(x, e, pe, edge_index, W_pe, b_pe, W1e, b1e, W2e, b2e,
           A, B, C, U, V, Wp1, bp1, Wp2, bp2):
    del x
    src = edge_index[0]
    dst = edge_index[1]
    srcg = jnp.concatenate([src, src + N]).reshape(2 * ECH, CK)
    dstg = jnp.concatenate([dst, dst + N]).reshape(2 * ECH, CK)
    dstl = dst.reshape(ECH, CK)
    srcl = src.reshape(ECH, CK)
    zer_np = jnp.zeros((NP, 2 * H), F32)
    zer_n = jnp.zeros((N, D), F32)

    bn = 2000
    nb_n = N // bn
    bk = 4000
    nb_e = E // bk

    h = pl.pallas_call(
        _enc_h_body,
        grid=(nb_n,),
        in_specs=[
            pl.BlockSpec((bn, 18), lambda i: (i, 0)),
            _full((18, D)),
            _full((1, D)),
        ],
        out_specs=pl.BlockSpec((bn, D), lambda i: (i, 0)),
        out_shape=jax.ShapeDtypeStruct((N, D), F32),
    )(pe, W_pe, b_pe.reshape(1, D))

    et2, etc = pl.pallas_call(
        _enc_et_body,
        grid=(nb_e,),
        in_specs=[
            pl.BlockSpec((bk, 16), lambda i: (i, 0)),
            _full((16, 64)),
            _full((1, 64)),
            _full((64, D)),
            _full((1, D)),
            _full((D, D)),
        ],
        out_specs=[
            pl.BlockSpec((bk, D), lambda i: (i, 0)),
            pl.BlockSpec((2, bk, H), lambda i: (0, i, 0)),
        ],
        out_shape=[
            jax.ShapeDtypeStruct((E, D), F32),
            jax.ShapeDtypeStruct((2, E, H), F32),
        ],
    )(e, W1e, b1e.reshape(1, 64), W2e, b2e.reshape(1, D), C[0])

    etl = zer_n
    for l in range(L):
        if l > 0:
            et2, etc = pl.pallas_call(
                _etc_body,
                grid=(nb_e,),
                in_specs=[
                    pl.BlockSpec((bk, D), lambda i: (i, 0)),
                    pl.BlockSpec((2, bk, H), lambda i: (0, i, 0)),
                    _full((D, D)),
                ],
                out_specs=[
                    pl.BlockSpec((bk, D), lambda i: (i, 0)),
                    pl.BlockSpec((2, bk, H), lambda i: (0, i, 0)),
                ],
                out_shape=[
                    jax.ShapeDtypeStruct((E, D), F32),
                    jax.ShapeDtypeStruct((2, E, H), F32),
                ],
            )(et2, r_split, C[l])

        hav, hb2, hu, gl, aggl, etl = pl.pallas_call(
            _tables_body,
            grid=(nb_n,),
            in_specs=[
                pl.BlockSpec((bn, D), lambda i: (i, 0)),
                pl.BlockSpec((bn, D), lambda i: (i, 0)),
                _full((D, D)), _full((D, D)), _full((D, D)),
                _full((D, D)), _full((D, D)),
            ],
            out_specs=[
                pl.BlockSpec((2, bn, D), lambda i: (0, i, 0)),
                pl.BlockSpec((2, bn, H), lambda i: (0, i, 0)),
                pl.BlockSpec((bn, D), lambda i: (i, 0)),
                pl.BlockSpec((bn, D), lambda i: (i, 0)),
                pl.BlockSpec((bn, D), lambda i: (i, 0)),
                pl.BlockSpec((bn, D), lambda i: (i, 0)),
            ],
            out_shape=[
                jax.ShapeDtypeStruct((2, N, D), F32),
                jax.ShapeDtypeStruct((2, N, H), F32),
                jax.ShapeDtypeStruct((N, D), F32),
                jax.ShapeDtypeStruct((N, D), F32),
                jax.ShapeDtypeStruct((N, D), F32),
                jax.ShapeDtypeStruct((N, D), F32),
            ],
        )(h, etl, A[l], B[l], V[l], U[l], C[l])

        r_split, acc2 = _edge_phase(
            srcg, dstg, dstl,
            hav.reshape(2 * N, D),
            hb2.reshape(2 * N, H),
            etc,
            zer_np,
        )

        h = pl.pallas_call(
            _hupd_body,
            grid=(nb_n,),
            in_specs=[
                pl.BlockSpec((bn, D), lambda i: (i, 0)),
                pl.BlockSpec((bn, D), lambda i: (i, 0)),
                pl.BlockSpec((bn, D), lambda i: (i, 0)),
                pl.BlockSpec((bn, D), lambda i: (i, 0)),
                pl.BlockSpec((2, bn, 2 * H), lambda i: (0, i, 0)),
            ],
            out_specs=pl.BlockSpec((bn, D), lambda i: (i, 0)),
            out_shape=jax.ShapeDtypeStruct((N, D), F32),
        )(h, hu, aggl, gl, acc2)

    hs1, hd1 = pl.pallas_call(
        _prednode_body,
        grid=(nb_n,),
        in_specs=[
            pl.BlockSpec((bn, D), lambda i: (i, 0)),
            _full((D, H)),
            _full((D, H)),
        ],
        out_specs=[
            pl.BlockSpec((bn, H), lambda i: (i, 0)),
            pl.BlockSpec((bn, H), lambda i: (i, 0)),
        ],
        out_shape=[
            jax.ShapeDtypeStruct((N, H), F32),
            jax.ShapeDtypeStruct((N, H), F32),
        ],
    )(h, Wp1[:D], Wp1[D:2 * D])

    etp = pl.pallas_call(
        _etp_body,
        grid=(nb_e,),
        in_specs=[
            pl.BlockSpec((bk, D), lambda i: (i, 0)),
            pl.BlockSpec((2, bk, H), lambda i: (0, i, 0)),
            _full((D, H)),
            _full((1, H)),
        ],
        out_specs=pl.BlockSpec((bk, H), lambda i: (i, 0)),
        out_shape=jax.ShapeDtypeStruct((E, H), F32),
    )(et2, r_split, Wp1[2 * D:], bp1.reshape(1, H))

    scores = _pred_phase(srcl, dstl, hs1, hd1, etp, Wp2.reshape(H))
    return scores.reshape(E, 1) + bp2
